# Initial kernel scaffold; baseline (speedup 1.0000x reference)
import jax
import jax.numpy as jnp
from jax.experimental import pallas as pl


def kernel(x, edge_index, W1, b1, W2, b2, W3, b3):
    raise NotImplementedError("write your pallas kernel here")


# speedup vs baseline: 12.8688x; 12.8688x over previous
import functools

import jax
import jax.numpy as jnp
from jax import lax
from jax.experimental import pallas as pl
from jax.experimental.pallas import tpu as pltpu
from jax.experimental.pallas import tpu_sc as plsc

N = 10000
E = 320000
D = 128
H = 64
OUT = 10
OUTP = 16

NPAD = 10240
NTILES = 32
EPT = NPAD
EPAD = NTILES * EPT
CHUNK = 128
NCHUNKS = EPT // CHUNK
ROWS_PER_TILE = NPAD // 16
TRASH = N + 8


def _sc_scatter_fn(width):
  mesh = plsc.VectorSubcoreMesh(core_axis_name="c", subcore_axis_name="s")

  @functools.partial(
      pl.kernel,
      mesh=mesh,
      compiler_params=pltpu.CompilerParams(use_tc_tiling_on_sc=False),
      out_type=jax.ShapeDtypeStruct((2, NPAD, width), jnp.float32),
      scratch_types=[
          pltpu.VMEM((NCHUNKS, CHUNK), jnp.int32),
          pltpu.VMEM((NCHUNKS, 1, CHUNK), jnp.int32),
          pltpu.VMEM((CHUNK, width), jnp.float32),
          pltpu.VMEM_SHARED((NPAD, width), jnp.float32),
          pltpu.SemaphoreType.DMA,
      ],
  )
  def sc_scatter(table_hbm, src_hbm, dst_hbm, zeros_hbm, out_hbm,
                 src_v, dst_v, rows_v, acc_sh, sem):
    c = lax.axis_index("c")
    s = lax.axis_index("s")
    wid = s * 2 + c
    base = s * ROWS_PER_TILE
    pltpu.sync_copy(zeros_hbm, acc_sh.at[pl.ds(base, ROWS_PER_TILE)])
    pltpu.sync_copy(src_hbm.at[wid], src_v)
    pltpu.sync_copy(dst_hbm.at[wid], dst_v)
    plsc.subcore_barrier()

    def body(j, carry):
      pltpu.async_copy(table_hbm.at[src_v.at[j]], rows_v, sem).wait()
      pltpu.sync_copy(rows_v, acc_sh.at[dst_v.at[j, 0]], add=True)
      return carry

    lax.fori_loop(0, NCHUNKS, body, 0)
    plsc.subcore_barrier()
    pltpu.sync_copy(acc_sh.at[pl.ds(base, ROWS_PER_TILE)],
                    out_hbm.at[c, pl.ds(base, ROWS_PER_TILE)])

  return sc_scatter


_sc_scatter_w64 = _sc_scatter_fn(H)
_sc_scatter_w16 = _sc_scatter_fn(OUTP)


def _dis(degp_ref):
  deg = degp_ref[0, :, 0:1] + degp_ref[1, :, 0:1] + 1.0
  return lax.rsqrt(deg)


def _tc_first_body(degp_ref, x_ref, w_ref, y_ref):
  dis = _dis(degp_ref)
  xw = jnp.dot(x_ref[...], w_ref[...], preferred_element_type=jnp.float32)
  y_ref[...] = xw * dis


def _tc_mid_body(degp_ref, acc_ref, y_ref, b_ref, w_ref, out_ref):
  dis = _dis(degp_ref)
  agg = acc_ref[0] + acc_ref[1] + y_ref[...]
  h = jnp.maximum(agg * dis + b_ref[...], 0.0)
  out_ref[...] = jnp.dot(h, w_ref[...], preferred_element_type=jnp.float32) * dis


def _tc_last_body(degp_ref, acc_ref, y_ref, b_ref, out_ref):
  dis = _dis(degp_ref)
  agg = acc_ref[0] + acc_ref[1] + y_ref[...]
  out_ref[...] = agg * dis + b_ref[...]


BM = 512
GRID = NPAD // BM


def _degp_spec():
  return pl.BlockSpec((2, BM, OUTP), lambda i: (0, i, 0))


def _tc_first(degp, x, w):
  return pl.pallas_call(
      _tc_first_body,
      grid=(GRID,),
      in_specs=[
          _degp_spec(),
          pl.BlockSpec((BM, D), lambda i: (i, 0)),
          pl.BlockSpec((D, H), lambda i: (0, 0)),
      ],
      out_specs=pl.BlockSpec((BM, H), lambda i: (i, 0)),
      out_shape=jax.ShapeDtypeStruct((NPAD, H), jnp.float32),
  )(degp, x, w)


def _tc_mid(degp, acc, y, b, w, wout):
  hin = y.shape[1]
  return pl.pallas_call(
      _tc_mid_body,
      grid=(GRID,),
      in_specs=[
          _degp_spec(),
          pl.BlockSpec((2, BM, hin), lambda i: (0, i, 0)),
          pl.BlockSpec((BM, hin), lambda i: (i, 0)),
          pl.BlockSpec((1, hin), lambda i: (0, 0)),
          pl.BlockSpec((hin, wout), lambda i: (0, 0)),
      ],
      out_specs=pl.BlockSpec((BM, wout), lambda i: (i, 0)),
      out_shape=jax.ShapeDtypeStruct((NPAD, wout), jnp.float32),
  )(degp, acc, y, b, w)


def _tc_last(degp, acc, y, b):
  hin = y.shape[1]
  return pl.pallas_call(
      _tc_last_body,
      grid=(GRID,),
      in_specs=[
          _degp_spec(),
          pl.BlockSpec((2, BM, hin), lambda i: (0, i, 0)),
          pl.BlockSpec((BM, hin), lambda i: (i, 0)),
          pl.BlockSpec((1, hin), lambda i: (0, 0)),
      ],
      out_specs=pl.BlockSpec((BM, hin), lambda i: (i, 0)),
      out_shape=jax.ShapeDtypeStruct((NPAD, hin), jnp.float32),
  )(degp, acc, y, b)


def kernel(x, edge_index, W1, b1, W2, b2, W3, b3):
  src = edge_index[0]
  dst = edge_index[1]
  pad_e = EPAD - E
  src_r = jnp.concatenate([src, jnp.zeros((pad_e,), jnp.int32)])
  src_r = src_r.reshape(NTILES, NCHUNKS, CHUNK)
  dst_r = jnp.concatenate([dst, jnp.full((pad_e,), TRASH, jnp.int32)])
  dst_r = dst_r.reshape(NTILES, NCHUNKS, 1, CHUNK)

  xp = jnp.pad(x, ((0, NPAD - N), (0, 0)))
  ones16 = jnp.ones((NPAD, OUTP), jnp.float32)
  z64 = jnp.zeros((ROWS_PER_TILE, H), jnp.float32)
  z16 = jnp.zeros((ROWS_PER_TILE, OUTP), jnp.float32)

  degp = _sc_scatter_w16(ones16, src_r, dst_r, z16)

  y1 = _tc_first(degp, xp, W1)
  acc1 = _sc_scatter_w64(y1, src_r, dst_r, z64)
  y2 = _tc_mid(degp, acc1, y1, b1.reshape(1, H), W2, H)
  acc2 = _sc_scatter_w64(y2, src_r, dst_r, z64)
  w3p = jnp.pad(W3, ((0, 0), (0, OUTP - OUT)))
  y3 = _tc_mid(degp, acc2, y2, b2.reshape(1, H), w3p, OUTP)
  acc3 = _sc_scatter_w16(y3, src_r, dst_r, z16)
  b3p = jnp.pad(b3, (0, OUTP - OUT)).reshape(1, OUTP)
  out = _tc_last(degp, acc3, y3, b3p)
  return out[:N, :OUT]


# speedup vs baseline: 17.2944x; 1.3439x over previous
import functools

import jax
import jax.numpy as jnp
from jax import lax
from jax.experimental import pallas as pl
from jax.experimental.pallas import tpu as pltpu
from jax.experimental.pallas import tpu_sc as plsc

N = 10000
E = 320000
D = 128
H = 64
OUT = 10
OUTP = 16

NPAD = 10240
NTILES = 32
EPT = NPAD
EPAD = NTILES * EPT
CHUNK = 128
NCHUNKS = EPT // CHUNK
ROWS_PER_TILE = NPAD // 16
TRASH = N + 8


def _sc_scatter_fn(width):
  mesh = plsc.VectorSubcoreMesh(core_axis_name="c", subcore_axis_name="s")

  @functools.partial(
      pl.kernel,
      mesh=mesh,
      compiler_params=pltpu.CompilerParams(use_tc_tiling_on_sc=False),
      out_type=jax.ShapeDtypeStruct((2, NPAD, width), jnp.float32),
      scratch_types=[
          pltpu.VMEM((NCHUNKS, CHUNK), jnp.int32),
          pltpu.VMEM((NCHUNKS, 1, CHUNK), jnp.int32),
          pltpu.VMEM((2, CHUNK, width), jnp.float32),
          pltpu.VMEM_SHARED((NPAD, width), jnp.float32),
          pltpu.SemaphoreType.DMA,
      ],
  )
  def sc_scatter(table_hbm, src_hbm, dst_hbm, zeros_hbm, out_hbm,
                 src_v, dst_v, bufs, acc_sh, sem):
    c = lax.axis_index("c")
    s = lax.axis_index("s")
    wid = s * 2 + c
    base = s * ROWS_PER_TILE
    pltpu.sync_copy(zeros_hbm, acc_sh.at[pl.ds(base, ROWS_PER_TILE)])
    pltpu.sync_copy(src_hbm.at[wid], src_v)
    pltpu.sync_copy(dst_hbm.at[wid], dst_v)
    plsc.subcore_barrier()

    pltpu.async_copy(table_hbm.at[src_v.at[0]], bufs.at[0], sem)

    def body(j, carry):
      par = lax.rem(j, 2)

      @pl.when(j < NCHUNKS - 1)
      def _():
        pltpu.async_copy(table_hbm.at[src_v.at[j + 1]], bufs.at[1 - par], sem)

      pltpu.make_async_copy(table_hbm.at[src_v.at[j]], bufs.at[par], sem).wait()
      pltpu.sync_copy(bufs.at[par], acc_sh.at[dst_v.at[j, 0]], add=True)
      return carry

    lax.fori_loop(0, NCHUNKS, body, 0)
    plsc.subcore_barrier()
    pltpu.sync_copy(acc_sh.at[pl.ds(base, ROWS_PER_TILE)],
                    out_hbm.at[c, pl.ds(base, ROWS_PER_TILE)])

  return sc_scatter


def _sc_deg_fn(width):
  mesh = plsc.VectorSubcoreMesh(core_axis_name="c", subcore_axis_name="s")

  @functools.partial(
      pl.kernel,
      mesh=mesh,
      compiler_params=pltpu.CompilerParams(use_tc_tiling_on_sc=False),
      out_type=jax.ShapeDtypeStruct((2, NPAD, width), jnp.float32),
      scratch_types=[
          pltpu.VMEM((NCHUNKS, 1, CHUNK), jnp.int32),
          pltpu.VMEM((CHUNK, width), jnp.float32),
          pltpu.VMEM_SHARED((NPAD, width), jnp.float32),
      ],
  )
  def sc_deg(ones_hbm, dst_hbm, zeros_hbm, out_hbm, dst_v, ones_v, acc_sh):
    c = lax.axis_index("c")
    s = lax.axis_index("s")
    wid = s * 2 + c
    base = s * ROWS_PER_TILE
    pltpu.sync_copy(zeros_hbm, acc_sh.at[pl.ds(base, ROWS_PER_TILE)])
    pltpu.sync_copy(ones_hbm, ones_v)
    pltpu.sync_copy(dst_hbm.at[wid], dst_v)
    plsc.subcore_barrier()

    def body(j, carry):
      pltpu.sync_copy(ones_v, acc_sh.at[dst_v.at[j, 0]], add=True)
      return carry

    lax.fori_loop(0, NCHUNKS, body, 0)
    plsc.subcore_barrier()
    pltpu.sync_copy(acc_sh.at[pl.ds(base, ROWS_PER_TILE)],
                    out_hbm.at[c, pl.ds(base, ROWS_PER_TILE)])

  return sc_deg


_sc_scatter_w64 = _sc_scatter_fn(H)
_sc_scatter_w16 = _sc_scatter_fn(OUTP)
_sc_deg_w16 = _sc_deg_fn(OUTP)


def _dis(degp_ref):
  deg = degp_ref[0, :, 0:1] + degp_ref[1, :, 0:1] + 1.0
  return lax.rsqrt(deg)


def _tc_first_body(degp_ref, x_ref, w_ref, y_ref):
  dis = _dis(degp_ref)
  xw = jnp.dot(x_ref[...], w_ref[...], preferred_element_type=jnp.float32)
  y_ref[...] = xw * dis


def _tc_mid_body(degp_ref, acc_ref, y_ref, b_ref, w_ref, out_ref):
  dis = _dis(degp_ref)
  agg = acc_ref[0] + acc_ref[1] + y_ref[...]
  h = jnp.maximum(agg * dis + b_ref[...], 0.0)
  out_ref[...] = jnp.dot(h, w_ref[...], preferred_element_type=jnp.float32) * dis


def _tc_last_body(degp_ref, acc_ref, y_ref, b_ref, out_ref):
  dis = _dis(degp_ref)
  agg = acc_ref[0] + acc_ref[1] + y_ref[...]
  out_ref[...] = agg * dis + b_ref[...]


BM = 512
GRID = NPAD // BM


def _degp_spec():
  return pl.BlockSpec((2, BM, OUTP), lambda i: (0, i, 0))


def _tc_first(degp, x, w):
  return pl.pallas_call(
      _tc_first_body,
      grid=(GRID,),
      in_specs=[
          _degp_spec(),
          pl.BlockSpec((BM, D), lambda i: (i, 0)),
          pl.BlockSpec((D, H), lambda i: (0, 0)),
      ],
      out_specs=pl.BlockSpec((BM, H), lambda i: (i, 0)),
      out_shape=jax.ShapeDtypeStruct((NPAD, H), jnp.float32),
  )(degp, x, w)


def _tc_mid(degp, acc, y, b, w, wout):
  hin = y.shape[1]
  return pl.pallas_call(
      _tc_mid_body,
      grid=(GRID,),
      in_specs=[
          _degp_spec(),
          pl.BlockSpec((2, BM, hin), lambda i: (0, i, 0)),
          pl.BlockSpec((BM, hin), lambda i: (i, 0)),
          pl.BlockSpec((1, hin), lambda i: (0, 0)),
          pl.BlockSpec((hin, wout), lambda i: (0, 0)),
      ],
      out_specs=pl.BlockSpec((BM, wout), lambda i: (i, 0)),
      out_shape=jax.ShapeDtypeStruct((NPAD, wout), jnp.float32),
  )(degp, acc, y, b, w)


def _tc_last(degp, acc, y, b):
  hin = y.shape[1]
  return pl.pallas_call(
      _tc_last_body,
      grid=(GRID,),
      in_specs=[
          _degp_spec(),
          pl.BlockSpec((2, BM, hin), lambda i: (0, i, 0)),
          pl.BlockSpec((BM, hin), lambda i: (i, 0)),
          pl.BlockSpec((1, hin), lambda i: (0, 0)),
      ],
      out_specs=pl.BlockSpec((BM, hin), lambda i: (i, 0)),
      out_shape=jax.ShapeDtypeStruct((NPAD, hin), jnp.float32),
  )(degp, acc, y, b)


def kernel(x, edge_index, W1, b1, W2, b2, W3, b3):
  src = edge_index[0]
  dst = edge_index[1]
  pad_e = EPAD - E
  trash = N + jnp.arange(pad_e, dtype=jnp.int32) % (NPAD - N)
  src_p = jnp.concatenate([src, jnp.zeros((pad_e,), jnp.int32)])
  dst_p = jnp.concatenate([dst, trash])
  src_r = src_p.reshape(EPT, NTILES).T.reshape(NTILES, NCHUNKS, CHUNK)
  dst_r = dst_p.reshape(EPT, NTILES).T.reshape(NTILES, NCHUNKS, 1, CHUNK)

  xp = jnp.pad(x, ((0, NPAD - N), (0, 0)))
  ones16 = jnp.ones((CHUNK, OUTP), jnp.float32)
  z64 = jnp.zeros((ROWS_PER_TILE, H), jnp.float32)
  z16 = jnp.zeros((ROWS_PER_TILE, OUTP), jnp.float32)

  degp = _sc_deg_w16(ones16, dst_r, z16)

  y1 = _tc_first(degp, xp, W1)
  acc1 = _sc_scatter_w64(y1, src_r, dst_r, z64)
  y2 = _tc_mid(degp, acc1, y1, b1.reshape(1, H), W2, H)
  acc2 = _sc_scatter_w64(y2, src_r, dst_r, z64)
  w3p = jnp.pad(W3, ((0, 0), (0, OUTP - OUT)))
  y3 = _tc_mid(degp, acc2, y2, b2.reshape(1, H), w3p, OUTP)
  acc3 = _sc_scatter_w16(y3, src_r, dst_r, z16)
  b3p = jnp.pad(b3, (0, OUTP - OUT)).reshape(1, OUTP)
  out = _tc_last(degp, acc3, y3, b3p)
  return out[:N, :OUT]


# speedup vs baseline: 31.8492x; 1.8416x over previous
import functools

import jax
import jax.numpy as jnp
from jax import lax
from jax.experimental import pallas as pl
from jax.experimental.pallas import tpu as pltpu
from jax.experimental.pallas import tpu_sc as plsc

N = 10000
E = 320000
D = 128
H = 64
OUT = 10
OUTP = 16

NPAD = 10240
NTILES = 32
EPT = NPAD
EPAD = NTILES * EPT
CHUNK = 128
NCHUNKS = EPT // CHUNK
ROWS_PER_TILE = NPAD // 16
TRASH = N + 8


def _sc_scatter_fn(width):
  mesh = plsc.VectorSubcoreMesh(core_axis_name="c", subcore_axis_name="s")

  @functools.partial(
      pl.kernel,
      mesh=mesh,
      compiler_params=pltpu.CompilerParams(use_tc_tiling_on_sc=False),
      out_type=jax.ShapeDtypeStruct((2, NPAD, width), jnp.float32),
      scratch_types=[
          pltpu.VMEM((NCHUNKS, CHUNK), jnp.int32),
          pltpu.VMEM((NCHUNKS, 1, CHUNK), jnp.int32),
          pltpu.VMEM((2, CHUNK, width), jnp.float32),
          pltpu.VMEM_SHARED((NPAD, width), jnp.float32),
          pltpu.VMEM_SHARED((NPAD, width), jnp.float32),
          pltpu.SemaphoreType.DMA,
      ],
  )
  def sc_scatter(table_hbm, src_hbm, dst_hbm, zeros_hbm, out_hbm,
                 src_v, dst_v, bufs, tab_sh, acc_sh, sem):
    c = lax.axis_index("c")
    s = lax.axis_index("s")
    wid = s * 2 + c
    base = s * ROWS_PER_TILE
    pltpu.sync_copy(zeros_hbm, acc_sh.at[pl.ds(base, ROWS_PER_TILE)])
    pltpu.sync_copy(table_hbm.at[pl.ds(base, ROWS_PER_TILE)],
                    tab_sh.at[pl.ds(base, ROWS_PER_TILE)])
    pltpu.sync_copy(src_hbm.at[wid], src_v)
    pltpu.sync_copy(dst_hbm.at[wid], dst_v)
    plsc.subcore_barrier()

    pltpu.async_copy(tab_sh.at[src_v.at[0]], bufs.at[0], sem)

    def body(j, carry):
      par = lax.rem(j, 2)

      @pl.when(j < NCHUNKS - 1)
      def _():
        pltpu.async_copy(tab_sh.at[src_v.at[j + 1]], bufs.at[1 - par], sem)

      pltpu.make_async_copy(tab_sh.at[src_v.at[j]], bufs.at[par], sem).wait()
      pltpu.sync_copy(bufs.at[par], acc_sh.at[dst_v.at[j, 0]], add=True)
      return carry

    lax.fori_loop(0, NCHUNKS, body, 0)
    plsc.subcore_barrier()
    pltpu.sync_copy(acc_sh.at[pl.ds(base, ROWS_PER_TILE)],
                    out_hbm.at[c, pl.ds(base, ROWS_PER_TILE)])

  return sc_scatter


def _sc_deg_fn(width):
  mesh = plsc.VectorSubcoreMesh(core_axis_name="c", subcore_axis_name="s")

  @functools.partial(
      pl.kernel,
      mesh=mesh,
      compiler_params=pltpu.CompilerParams(use_tc_tiling_on_sc=False),
      out_type=jax.ShapeDtypeStruct((2, NPAD, width), jnp.float32),
      scratch_types=[
          pltpu.VMEM((NCHUNKS, 1, CHUNK), jnp.int32),
          pltpu.VMEM((CHUNK, width), jnp.float32),
          pltpu.VMEM_SHARED((NPAD, width), jnp.float32),
      ],
  )
  def sc_deg(ones_hbm, dst_hbm, zeros_hbm, out_hbm, dst_v, ones_v, acc_sh):
    c = lax.axis_index("c")
    s = lax.axis_index("s")
    wid = s * 2 + c
    base = s * ROWS_PER_TILE
    pltpu.sync_copy(zeros_hbm, acc_sh.at[pl.ds(base, ROWS_PER_TILE)])
    pltpu.sync_copy(ones_hbm, ones_v)
    pltpu.sync_copy(dst_hbm.at[wid], dst_v)
    plsc.subcore_barrier()

    def body(j, carry):
      pltpu.sync_copy(ones_v, acc_sh.at[dst_v.at[j, 0]], add=True)
      return carry

    lax.fori_loop(0, NCHUNKS, body, 0)
    plsc.subcore_barrier()
    pltpu.sync_copy(acc_sh.at[pl.ds(base, ROWS_PER_TILE)],
                    out_hbm.at[c, pl.ds(base, ROWS_PER_TILE)])

  return sc_deg


_sc_scatter_w64 = _sc_scatter_fn(H)
_sc_scatter_w16 = _sc_scatter_fn(OUTP)
_sc_deg_w16 = _sc_deg_fn(OUTP)


def _dis(degp_ref):
  deg = degp_ref[0, :, 0:1] + degp_ref[1, :, 0:1] + 1.0
  return lax.rsqrt(deg)


def _tc_first_body(degp_ref, x_ref, w_ref, y_ref):
  dis = _dis(degp_ref)
  xw = jnp.dot(x_ref[...], w_ref[...], preferred_element_type=jnp.float32)
  y_ref[...] = xw * dis


def _tc_mid_body(degp_ref, acc_ref, y_ref, b_ref, w_ref, out_ref):
  dis = _dis(degp_ref)
  agg = acc_ref[0] + acc_ref[1] + y_ref[...]
  h = jnp.maximum(agg * dis + b_ref[...], 0.0)
  out_ref[...] = jnp.dot(h, w_ref[...], preferred_element_type=jnp.float32) * dis


def _tc_last_body(degp_ref, acc_ref, y_ref, b_ref, out_ref):
  dis = _dis(degp_ref)
  agg = acc_ref[0] + acc_ref[1] + y_ref[...]
  out_ref[...] = agg * dis + b_ref[...]


BM = 512
GRID = NPAD // BM


def _degp_spec():
  return pl.BlockSpec((2, BM, OUTP), lambda i: (0, i, 0))


def _tc_first(degp, x, w):
  return pl.pallas_call(
      _tc_first_body,
      grid=(GRID,),
      in_specs=[
          _degp_spec(),
          pl.BlockSpec((BM, D), lambda i: (i, 0)),
          pl.BlockSpec((D, H), lambda i: (0, 0)),
      ],
      out_specs=pl.BlockSpec((BM, H), lambda i: (i, 0)),
      out_shape=jax.ShapeDtypeStruct((NPAD, H), jnp.float32),
  )(degp, x, w)


def _tc_mid(degp, acc, y, b, w, wout):
  hin = y.shape[1]
  return pl.pallas_call(
      _tc_mid_body,
      grid=(GRID,),
      in_specs=[
          _degp_spec(),
          pl.BlockSpec((2, BM, hin), lambda i: (0, i, 0)),
          pl.BlockSpec((BM, hin), lambda i: (i, 0)),
          pl.BlockSpec((1, hin), lambda i: (0, 0)),
          pl.BlockSpec((hin, wout), lambda i: (0, 0)),
      ],
      out_specs=pl.BlockSpec((BM, wout), lambda i: (i, 0)),
      out_shape=jax.ShapeDtypeStruct((NPAD, wout), jnp.float32),
  )(degp, acc, y, b, w)


def _tc_last(degp, acc, y, b):
  hin = y.shape[1]
  return pl.pallas_call(
      _tc_last_body,
      grid=(GRID,),
      in_specs=[
          _degp_spec(),
          pl.BlockSpec((2, BM, hin), lambda i: (0, i, 0)),
          pl.BlockSpec((BM, hin), lambda i: (i, 0)),
          pl.BlockSpec((1, hin), lambda i: (0, 0)),
      ],
      out_specs=pl.BlockSpec((BM, hin), lambda i: (i, 0)),
      out_shape=jax.ShapeDtypeStruct((NPAD, hin), jnp.float32),
  )(degp, acc, y, b)


def kernel(x, edge_index, W1, b1, W2, b2, W3, b3):
  src = edge_index[0]
  dst = edge_index[1]
  pad_e = EPAD - E
  trash = N + jnp.arange(pad_e, dtype=jnp.int32) % (NPAD - N)
  src_p = jnp.concatenate([src, jnp.zeros((pad_e,), jnp.int32)])
  dst_p = jnp.concatenate([dst, trash])
  src_r = src_p.reshape(EPT, NTILES).T.reshape(NTILES, NCHUNKS, CHUNK)
  dst_r = dst_p.reshape(EPT, NTILES).T.reshape(NTILES, NCHUNKS, 1, CHUNK)

  xp = jnp.pad(x, ((0, NPAD - N), (0, 0)))
  ones16 = jnp.ones((CHUNK, OUTP), jnp.float32)
  z64 = jnp.zeros((ROWS_PER_TILE, H), jnp.float32)
  z16 = jnp.zeros((ROWS_PER_TILE, OUTP), jnp.float32)

  degp = _sc_deg_w16(ones16, dst_r, z16)

  y1 = _tc_first(degp, xp, W1)
  acc1 = _sc_scatter_w64(y1, src_r, dst_r, z64)
  y2 = _tc_mid(degp, acc1, y1, b1.reshape(1, H), W2, H)
  acc2 = _sc_scatter_w64(y2, src_r, dst_r, z64)
  w3p = jnp.pad(W3, ((0, 0), (0, OUTP - OUT)))
  y3 = _tc_mid(degp, acc2, y2, b2.reshape(1, H), w3p, OUTP)
  acc3 = _sc_scatter_w16(y3, src_r, dst_r, z16)
  b3p = jnp.pad(b3, (0, OUTP - OUT)).reshape(1, OUTP)
  out = _tc_last(degp, acc3, y3, b3p)
  return out[:N, :OUT]


# speedup vs baseline: 32.4028x; 1.0174x over previous
import functools

import jax
import jax.numpy as jnp
from jax import lax
from jax.experimental import pallas as pl
from jax.experimental.pallas import tpu as pltpu
from jax.experimental.pallas import tpu_sc as plsc

N = 10000
E = 320000
D = 128
H = 64
OUT = 10
OUTP = 16

NPAD = 10240
NTILES = 32
EPT = NPAD
EPAD = NTILES * EPT
CHUNK = 128
NCHUNKS = EPT // CHUNK
ROWS_PER_TILE = NPAD // 16
TRASH = N + 8


def _sc_scatter_fn(width):
  mesh = plsc.VectorSubcoreMesh(core_axis_name="c", subcore_axis_name="s")

  @functools.partial(
      pl.kernel,
      mesh=mesh,
      compiler_params=pltpu.CompilerParams(use_tc_tiling_on_sc=False),
      out_type=jax.ShapeDtypeStruct((2, NPAD, width), jnp.float32),
      scratch_types=[
          pltpu.VMEM((NCHUNKS, CHUNK), jnp.int32),
          pltpu.VMEM((NCHUNKS, 1, CHUNK), jnp.int32),
          pltpu.VMEM((2, CHUNK, width), jnp.float32),
          pltpu.VMEM_SHARED((NPAD, width), jnp.float32),
          pltpu.VMEM_SHARED((NPAD, width), jnp.float32),
          pltpu.SemaphoreType.DMA,
      ],
  )
  def sc_scatter(table_hbm, src_hbm, dst_hbm, zeros_hbm, out_hbm,
                 src_v, dst_v, bufs, tab_sh, acc_sh, sem):
    c = lax.axis_index("c")
    s = lax.axis_index("s")
    wid = s * 2 + c
    base = s * ROWS_PER_TILE
    pltpu.sync_copy(zeros_hbm, acc_sh.at[pl.ds(base, ROWS_PER_TILE)])
    pltpu.sync_copy(table_hbm.at[pl.ds(base, ROWS_PER_TILE)],
                    tab_sh.at[pl.ds(base, ROWS_PER_TILE)])
    pltpu.sync_copy(src_hbm.at[wid], src_v)
    pltpu.sync_copy(dst_hbm.at[wid], dst_v)
    plsc.subcore_barrier()

    pltpu.async_copy(tab_sh.at[src_v.at[0]], bufs.at[0], sem)

    def body(j, carry):
      par = lax.rem(j, 2)

      @pl.when(j < NCHUNKS - 1)
      def _():
        pltpu.async_copy(tab_sh.at[src_v.at[j + 1]], bufs.at[1 - par], sem)

      pltpu.make_async_copy(tab_sh.at[src_v.at[j]], bufs.at[par], sem).wait()
      pltpu.sync_copy(bufs.at[par], acc_sh.at[dst_v.at[j, 0]], add=True)
      return carry

    lax.fori_loop(0, NCHUNKS, body, 0)
    plsc.subcore_barrier()
    pltpu.sync_copy(acc_sh.at[pl.ds(base, ROWS_PER_TILE)],
                    out_hbm.at[c, pl.ds(base, ROWS_PER_TILE)])

  return sc_scatter


def _sc_deg_fn(width):
  mesh = plsc.VectorSubcoreMesh(core_axis_name="c", subcore_axis_name="s")

  @functools.partial(
      pl.kernel,
      mesh=mesh,
      compiler_params=pltpu.CompilerParams(use_tc_tiling_on_sc=False),
      out_type=jax.ShapeDtypeStruct((2, NPAD, width), jnp.float32),
      scratch_types=[
          pltpu.VMEM((NCHUNKS, 1, CHUNK), jnp.int32),
          pltpu.VMEM((CHUNK, width), jnp.float32),
          pltpu.VMEM_SHARED((NPAD, width), jnp.float32),
      ],
  )
  def sc_deg(ones_hbm, dst_hbm, zeros_hbm, out_hbm, dst_v, ones_v, acc_sh):
    c = lax.axis_index("c")
    s = lax.axis_index("s")
    wid = s * 2 + c
    base = s * ROWS_PER_TILE
    pltpu.sync_copy(zeros_hbm, acc_sh.at[pl.ds(base, ROWS_PER_TILE)])
    pltpu.sync_copy(ones_hbm, ones_v)
    pltpu.sync_copy(dst_hbm.at[wid], dst_v)
    plsc.subcore_barrier()

    def body(j, carry):
      pltpu.sync_copy(ones_v, acc_sh.at[dst_v.at[j, 0]], add=True)
      return carry

    lax.fori_loop(0, NCHUNKS, body, 0)
    plsc.subcore_barrier()
    pltpu.sync_copy(acc_sh.at[pl.ds(base, ROWS_PER_TILE)],
                    out_hbm.at[c, pl.ds(base, ROWS_PER_TILE)])

  return sc_deg


_sc_scatter_w64 = _sc_scatter_fn(H)
_sc_scatter_w16 = _sc_scatter_fn(OUTP)
_sc_deg_w16 = _sc_deg_fn(OUTP)


def _dis(degp_ref):
  deg = degp_ref[0, :, 0:1] + degp_ref[1, :, 0:1] + 1.0
  return lax.rsqrt(deg)


def _tc_first_body(degp_ref, x_ref, w_ref, y_ref):
  dis = _dis(degp_ref)
  xw = jnp.dot(x_ref[...], w_ref[...], preferred_element_type=jnp.float32)
  y_ref[...] = xw * dis


def _tc_mid_body(degp_ref, acc_ref, y_ref, b_ref, w_ref, out_ref):
  dis = _dis(degp_ref)
  agg = acc_ref[0] + acc_ref[1] + y_ref[...]
  h = jnp.maximum(agg * dis + b_ref[...], 0.0)
  out_ref[...] = jnp.dot(h, w_ref[...], preferred_element_type=jnp.float32) * dis


def _tc_last_body(degp_ref, acc_ref, y_ref, b_ref, out_ref):
  dis = _dis(degp_ref)
  agg = acc_ref[0] + acc_ref[1] + y_ref[...]
  out_ref[...] = agg * dis + b_ref[...]


BM = 512
GRID = NPAD // BM


def _degp_spec():
  return pl.BlockSpec((2, BM, OUTP), lambda i: (0, i, 0))


def _tc_first(degp, x, w):
  return pl.pallas_call(
      _tc_first_body,
      grid=(GRID,),
      in_specs=[
          _degp_spec(),
          pl.BlockSpec((BM, D), lambda i: (i, 0)),
          pl.BlockSpec((D, H), lambda i: (0, 0)),
      ],
      out_specs=pl.BlockSpec((BM, H), lambda i: (i, 0)),
      out_shape=jax.ShapeDtypeStruct((NPAD, H), jnp.float32),
  )(degp, x, w)


def _tc_mid(degp, acc, y, b, w, wout):
  hin = y.shape[1]
  return pl.pallas_call(
      _tc_mid_body,
      grid=(GRID,),
      in_specs=[
          _degp_spec(),
          pl.BlockSpec((2, BM, hin), lambda i: (0, i, 0)),
          pl.BlockSpec((BM, hin), lambda i: (i, 0)),
          pl.BlockSpec((1, hin), lambda i: (0, 0)),
          pl.BlockSpec((hin, wout), lambda i: (0, 0)),
      ],
      out_specs=pl.BlockSpec((BM, wout), lambda i: (i, 0)),
      out_shape=jax.ShapeDtypeStruct((NPAD, wout), jnp.float32),
  )(degp, acc, y, b, w)


def _tc_last(degp, acc, y, b):
  hin = y.shape[1]
  return pl.pallas_call(
      _tc_last_body,
      grid=(GRID,),
      in_specs=[
          _degp_spec(),
          pl.BlockSpec((2, BM, hin), lambda i: (0, i, 0)),
          pl.BlockSpec((BM, hin), lambda i: (i, 0)),
          pl.BlockSpec((1, hin), lambda i: (0, 0)),
      ],
      out_specs=pl.BlockSpec((BM, hin), lambda i: (i, 0)),
      out_shape=jax.ShapeDtypeStruct((NPAD, hin), jnp.float32),
  )(degp, acc, y, b)


def kernel(x, edge_index, W1, b1, W2, b2, W3, b3):
  src = edge_index[0]
  dst = edge_index[1]
  pad_e = EPAD - E
  trash = N + jnp.arange(pad_e, dtype=jnp.int32) % (NPAD - N)
  src_p = jnp.concatenate([src, jnp.zeros((pad_e,), jnp.int32)])
  dst_p = jnp.concatenate([dst, trash])
  src_r = src_p.reshape(NTILES, NCHUNKS, CHUNK)
  dst_r = dst_p.reshape(NTILES, NCHUNKS, 1, CHUNK)

  xp = jnp.pad(x, ((0, NPAD - N), (0, 0)))
  ones16 = jnp.ones((CHUNK, OUTP), jnp.float32)
  z64 = jnp.zeros((ROWS_PER_TILE, H), jnp.float32)
  z16 = jnp.zeros((ROWS_PER_TILE, OUTP), jnp.float32)

  degp = _sc_deg_w16(ones16, dst_r, z16)

  y1 = _tc_first(degp, xp, W1)
  acc1 = _sc_scatter_w64(y1, src_r, dst_r, z64)
  y2 = _tc_mid(degp, acc1, y1, b1.reshape(1, H), W2, H)
  acc2 = _sc_scatter_w64(y2, src_r, dst_r, z64)
  w3p = jnp.pad(W3, ((0, 0), (0, OUTP - OUT)))
  y3 = _tc_mid(degp, acc2, y2, b2.reshape(1, H), w3p, OUTP)
  acc3 = _sc_scatter_w16(y3, src_r, dst_r, z16)
  b3p = jnp.pad(b3, (0, OUTP - OUT)).reshape(1, OUTP)
  out = _tc_last(degp, acc3, y3, b3p)
  return out[:N, :OUT]


# speedup vs baseline: 35.3624x; 1.0913x over previous
import functools

import jax
import jax.numpy as jnp
from jax import lax
from jax.experimental import pallas as pl
from jax.experimental.pallas import tpu as pltpu
from jax.experimental.pallas import tpu_sc as plsc

N = 10000
E = 320000
D = 128
H = 64
OUT = 10
OUTP = 16

NPAD = 10240
NTILES = 32
EPT = NPAD
EPAD = NTILES * EPT
CHUNK = 128
NCHUNKS = EPT // CHUNK
ROWS_PER_TILE = NPAD // 16
TRASH = N + 8


def _sc_scatter_fn(width):
  mesh = plsc.VectorSubcoreMesh(core_axis_name="c", subcore_axis_name="s")

  @functools.partial(
      pl.kernel,
      mesh=mesh,
      compiler_params=pltpu.CompilerParams(use_tc_tiling_on_sc=False),
      out_type=jax.ShapeDtypeStruct((2, NPAD, width), jnp.float32),
      scratch_types=[
          pltpu.VMEM((NCHUNKS, CHUNK), jnp.int32),
          pltpu.VMEM((NCHUNKS, 1, CHUNK), jnp.int32),
          pltpu.VMEM((2, CHUNK, width), jnp.float32),
          pltpu.VMEM_SHARED((NPAD, width), jnp.float32),
          pltpu.VMEM_SHARED((NPAD, width), jnp.float32),
          pltpu.SemaphoreType.DMA,
      ],
  )
  def sc_scatter(table_hbm, src_hbm, dst_hbm, zeros_hbm, out_hbm,
                 src_v, dst_v, bufs, tab_sh, acc_sh, sem):
    c = lax.axis_index("c")
    s = lax.axis_index("s")
    wid = s * 2 + c
    base = s * ROWS_PER_TILE
    pltpu.sync_copy(zeros_hbm, acc_sh.at[pl.ds(base, ROWS_PER_TILE)])
    pltpu.sync_copy(table_hbm.at[pl.ds(base, ROWS_PER_TILE)],
                    tab_sh.at[pl.ds(base, ROWS_PER_TILE)])
    pltpu.sync_copy(src_hbm.at[wid], src_v)
    pltpu.sync_copy(dst_hbm.at[wid], dst_v)
    plsc.subcore_barrier()

    pltpu.async_copy(tab_sh.at[src_v.at[0]], bufs.at[0], sem)

    def body(j, carry):
      par = lax.rem(j, 2)

      @pl.when(j < NCHUNKS - 1)
      def _():
        pltpu.async_copy(tab_sh.at[src_v.at[j + 1]], bufs.at[1 - par], sem)

      pltpu.make_async_copy(tab_sh.at[src_v.at[j]], bufs.at[par], sem).wait()
      pltpu.sync_copy(bufs.at[par], acc_sh.at[dst_v.at[j, 0]], add=True)
      return carry

    lax.fori_loop(0, NCHUNKS, body, 0)
    plsc.subcore_barrier()
    pltpu.sync_copy(acc_sh.at[pl.ds(base, ROWS_PER_TILE)],
                    out_hbm.at[c, pl.ds(base, ROWS_PER_TILE)])

  return sc_scatter


def _sc_deg_fn(width):
  mesh = plsc.VectorSubcoreMesh(core_axis_name="c", subcore_axis_name="s")

  @functools.partial(
      pl.kernel,
      mesh=mesh,
      compiler_params=pltpu.CompilerParams(use_tc_tiling_on_sc=False),
      out_type=jax.ShapeDtypeStruct((2, NPAD, width), jnp.float32),
      scratch_types=[
          pltpu.VMEM((NCHUNKS, 1, CHUNK), jnp.int32),
          pltpu.VMEM((CHUNK, width), jnp.float32),
          pltpu.VMEM_SHARED((NPAD, width), jnp.float32),
      ],
  )
  def sc_deg(ones_hbm, dst_hbm, zeros_hbm, out_hbm, dst_v, ones_v, acc_sh):
    c = lax.axis_index("c")
    s = lax.axis_index("s")
    wid = s * 2 + c
    base = s * ROWS_PER_TILE
    pltpu.sync_copy(zeros_hbm, acc_sh.at[pl.ds(base, ROWS_PER_TILE)])
    pltpu.sync_copy(ones_hbm, ones_v)
    pltpu.sync_copy(dst_hbm.at[wid], dst_v)
    plsc.subcore_barrier()

    def body(j, carry):
      pltpu.sync_copy(ones_v, acc_sh.at[dst_v.at[j, 0]], add=True)
      return carry

    lax.fori_loop(0, NCHUNKS, body, 0)
    plsc.subcore_barrier()
    pltpu.sync_copy(acc_sh.at[pl.ds(base, ROWS_PER_TILE)],
                    out_hbm.at[c, pl.ds(base, ROWS_PER_TILE)])

  return sc_deg


_sc_scatter_w64 = _sc_scatter_fn(H)
_sc_scatter_w16 = _sc_scatter_fn(OUTP)
_sc_deg_w16 = _sc_deg_fn(OUTP)


def _dis(degp_ref):
  deg = degp_ref[0, :, 0:1] + degp_ref[1, :, 0:1] + 1.0
  return lax.rsqrt(deg)


def _tc_first_body(degp_ref, x_ref, w_ref, y_ref):
  dis = _dis(degp_ref)
  xw = jnp.dot(x_ref[...], w_ref[...], preferred_element_type=jnp.float32)
  y_ref[...] = xw * dis


def _tc_mid_body(degp_ref, acc_ref, y_ref, b_ref, w_ref, out_ref):
  dis = _dis(degp_ref)
  agg = acc_ref[0] + acc_ref[1] + y_ref[...]
  h = jnp.maximum(agg * dis + b_ref[...], 0.0)
  out_ref[...] = jnp.dot(h, w_ref[...], preferred_element_type=jnp.float32) * dis


def _tc_last_body(degp_ref, acc_ref, y_ref, b_ref, out_ref):
  dis = _dis(degp_ref)
  agg = acc_ref[0] + acc_ref[1] + y_ref[...]
  out_ref[...] = agg * dis + b_ref[...]


BM = 2048
GRID = NPAD // BM


def _degp_spec():
  return pl.BlockSpec((2, BM, OUTP), lambda i: (0, i, 0))


def _tc_first(degp, x, w):
  return pl.pallas_call(
      _tc_first_body,
      grid=(GRID,),
      in_specs=[
          _degp_spec(),
          pl.BlockSpec((BM, D), lambda i: (i, 0)),
          pl.BlockSpec((D, H), lambda i: (0, 0)),
      ],
      out_specs=pl.BlockSpec((BM, H), lambda i: (i, 0)),
      out_shape=jax.ShapeDtypeStruct((NPAD, H), jnp.float32),
  )(degp, x, w)


def _tc_mid(degp, acc, y, b, w, wout):
  hin = y.shape[1]
  return pl.pallas_call(
      _tc_mid_body,
      grid=(GRID,),
      in_specs=[
          _degp_spec(),
          pl.BlockSpec((2, BM, hin), lambda i: (0, i, 0)),
          pl.BlockSpec((BM, hin), lambda i: (i, 0)),
          pl.BlockSpec((1, hin), lambda i: (0, 0)),
          pl.BlockSpec((hin, wout), lambda i: (0, 0)),
      ],
      out_specs=pl.BlockSpec((BM, wout), lambda i: (i, 0)),
      out_shape=jax.ShapeDtypeStruct((NPAD, wout), jnp.float32),
  )(degp, acc, y, b, w)


def _tc_last(degp, acc, y, b):
  hin = y.shape[1]
  return pl.pallas_call(
      _tc_last_body,
      grid=(GRID,),
      in_specs=[
          _degp_spec(),
          pl.BlockSpec((2, BM, hin), lambda i: (0, i, 0)),
          pl.BlockSpec((BM, hin), lambda i: (i, 0)),
          pl.BlockSpec((1, hin), lambda i: (0, 0)),
      ],
      out_specs=pl.BlockSpec((BM, hin), lambda i: (i, 0)),
      out_shape=jax.ShapeDtypeStruct((NPAD, hin), jnp.float32),
  )(degp, acc, y, b)


def kernel(x, edge_index, W1, b1, W2, b2, W3, b3):
  src = edge_index[0]
  dst = edge_index[1]
  pad_e = EPAD - E
  trash = N + jnp.arange(pad_e, dtype=jnp.int32) % (NPAD - N)
  src_p = jnp.concatenate([src, jnp.zeros((pad_e,), jnp.int32)])
  dst_p = jnp.concatenate([dst, trash])
  src_r = src_p.reshape(NTILES, NCHUNKS, CHUNK)
  dst_r = dst_p.reshape(NTILES, NCHUNKS, 1, CHUNK)

  xp = jnp.pad(x, ((0, NPAD - N), (0, 0)))
  ones16 = jnp.ones((CHUNK, OUTP), jnp.float32)
  z64 = jnp.zeros((ROWS_PER_TILE, H), jnp.float32)
  z16 = jnp.zeros((ROWS_PER_TILE, OUTP), jnp.float32)

  degp = _sc_deg_w16(ones16, dst_r, z16)

  y1 = _tc_first(degp, xp, W1)
  acc1 = _sc_scatter_w64(y1, src_r, dst_r, z64)
  y2 = _tc_mid(degp, acc1, y1, b1.reshape(1, H), W2, H)
  acc2 = _sc_scatter_w64(y2, src_r, dst_r, z64)
  w3p = jnp.pad(W3, ((0, 0), (0, OUTP - OUT)))
  y3 = _tc_mid(degp, acc2, y2, b2.reshape(1, H), w3p, OUTP)
  acc3 = _sc_scatter_w16(y3, src_r, dst_r, z16)
  b3p = jnp.pad(b3, (0, OUTP - OUT)).reshape(1, OUTP)
  out = _tc_last(degp, acc3, y3, b3p)
  return out[:N, :OUT]


# speedup vs baseline: 37.4969x; 1.0604x over previous
import functools

import jax
import jax.numpy as jnp
from jax import lax
from jax.experimental import pallas as pl
from jax.experimental.pallas import tpu as pltpu
from jax.experimental.pallas import tpu_sc as plsc

N = 10000
E = 320000
D = 128
H = 64
OUT = 10
OUTP = 16

NPAD = 10240
NTILES = 32
EPT = NPAD
EPAD = NTILES * EPT
CHUNK = 128
NCHUNKS = EPT // CHUNK
ROWS_PER_TILE = NPAD // 16
TRASH = N + 8


def _sc_scatter_fn(width):
  mesh = plsc.VectorSubcoreMesh(core_axis_name="c", subcore_axis_name="s")

  @functools.partial(
      pl.kernel,
      mesh=mesh,
      compiler_params=pltpu.CompilerParams(use_tc_tiling_on_sc=False),
      out_type=jax.ShapeDtypeStruct((2, NPAD, width), jnp.float32),
      scratch_types=[
          pltpu.VMEM((NCHUNKS, CHUNK), jnp.int32),
          pltpu.VMEM((NCHUNKS, 1, CHUNK), jnp.int32),
          pltpu.VMEM((3, CHUNK, width), jnp.float32),
          pltpu.VMEM_SHARED((NPAD, width), jnp.float32),
          pltpu.VMEM_SHARED((NPAD, width), jnp.float32),
          pltpu.SemaphoreType.DMA,
          pltpu.SemaphoreType.DMA,
      ],
  )
  def sc_scatter(table_hbm, src_hbm, dst_hbm, zeros_hbm, out_hbm,
                 src_v, dst_v, bufs, tab_sh, acc_sh, semg, sems):
    c = lax.axis_index("c")
    s = lax.axis_index("s")
    wid = s * 2 + c
    base = s * ROWS_PER_TILE
    pltpu.sync_copy(zeros_hbm, acc_sh.at[pl.ds(base, ROWS_PER_TILE)])
    pltpu.sync_copy(table_hbm.at[pl.ds(base, ROWS_PER_TILE)],
                    tab_sh.at[pl.ds(base, ROWS_PER_TILE)])
    pltpu.sync_copy(src_hbm.at[wid], src_v)
    pltpu.sync_copy(dst_hbm.at[wid], dst_v)
    plsc.subcore_barrier()

    pltpu.async_copy(tab_sh.at[src_v.at[0]], bufs.at[0], semg)
    pltpu.async_copy(tab_sh.at[src_v.at[1]], bufs.at[1], semg)

    def body(j, carry):
      par = lax.rem(j, 3)
      pltpu.make_async_copy(tab_sh.at[src_v.at[j]], bufs.at[par], semg).wait()
      pltpu.async_copy(bufs.at[par], acc_sh.at[dst_v.at[j, 0]], sems,
                       add=True)

      @pl.when(j >= 1)
      def _():
        pltpu.make_async_copy(bufs.at[par], acc_sh.at[dst_v.at[j, 0]],
                              sems).wait()

      @pl.when(j < NCHUNKS - 2)
      def _():
        pltpu.async_copy(tab_sh.at[src_v.at[j + 2]],
                         bufs.at[lax.rem(j + 2, 3)], semg)
      return carry

    lax.fori_loop(0, NCHUNKS, body, 0)
    pltpu.make_async_copy(bufs.at[0], acc_sh.at[dst_v.at[0, 0]], sems).wait()
    plsc.subcore_barrier()
    pltpu.sync_copy(acc_sh.at[pl.ds(base, ROWS_PER_TILE)],
                    out_hbm.at[c, pl.ds(base, ROWS_PER_TILE)])

  return sc_scatter


def _sc_deg_fn(width):
  mesh = plsc.VectorSubcoreMesh(core_axis_name="c", subcore_axis_name="s")

  @functools.partial(
      pl.kernel,
      mesh=mesh,
      compiler_params=pltpu.CompilerParams(use_tc_tiling_on_sc=False),
      out_type=jax.ShapeDtypeStruct((2, NPAD, width), jnp.float32),
      scratch_types=[
          pltpu.VMEM((NCHUNKS, 1, CHUNK), jnp.int32),
          pltpu.VMEM((CHUNK, width), jnp.float32),
          pltpu.VMEM_SHARED((NPAD, width), jnp.float32),
      ],
  )
  def sc_deg(ones_hbm, dst_hbm, zeros_hbm, out_hbm, dst_v, ones_v, acc_sh):
    c = lax.axis_index("c")
    s = lax.axis_index("s")
    wid = s * 2 + c
    base = s * ROWS_PER_TILE
    pltpu.sync_copy(zeros_hbm, acc_sh.at[pl.ds(base, ROWS_PER_TILE)])
    pltpu.sync_copy(ones_hbm, ones_v)
    pltpu.sync_copy(dst_hbm.at[wid], dst_v)
    plsc.subcore_barrier()

    def body(j, carry):
      pltpu.sync_copy(ones_v, acc_sh.at[dst_v.at[j, 0]], add=True)
      return carry

    lax.fori_loop(0, NCHUNKS, body, 0)
    plsc.subcore_barrier()
    pltpu.sync_copy(acc_sh.at[pl.ds(base, ROWS_PER_TILE)],
                    out_hbm.at[c, pl.ds(base, ROWS_PER_TILE)])

  return sc_deg


_sc_scatter_w64 = _sc_scatter_fn(H)
_sc_scatter_w16 = _sc_scatter_fn(OUTP)
_sc_deg_w16 = _sc_deg_fn(OUTP)


def _dis(degp_ref):
  deg = degp_ref[0, :, 0:1] + degp_ref[1, :, 0:1] + 1.0
  return lax.rsqrt(deg)


def _tc_first_body(degp_ref, x_ref, w_ref, y_ref):
  dis = _dis(degp_ref)
  xw = jnp.dot(x_ref[...], w_ref[...], preferred_element_type=jnp.float32)
  y_ref[...] = xw * dis


def _tc_mid_body(degp_ref, acc_ref, y_ref, b_ref, w_ref, out_ref):
  dis = _dis(degp_ref)
  agg = acc_ref[0] + acc_ref[1] + y_ref[...]
  h = jnp.maximum(agg * dis + b_ref[...], 0.0)
  out_ref[...] = jnp.dot(h, w_ref[...], preferred_element_type=jnp.float32) * dis


def _tc_last_body(degp_ref, acc_ref, y_ref, b_ref, out_ref):
  dis = _dis(degp_ref)
  agg = acc_ref[0] + acc_ref[1] + y_ref[...]
  out_ref[...] = agg * dis + b_ref[...]


BM = 2048
GRID = NPAD // BM


def _degp_spec():
  return pl.BlockSpec((2, BM, OUTP), lambda i: (0, i, 0))


def _tc_first(degp, x, w):
  return pl.pallas_call(
      _tc_first_body,
      grid=(GRID,),
      in_specs=[
          _degp_spec(),
          pl.BlockSpec((BM, D), lambda i: (i, 0)),
          pl.BlockSpec((D, H), lambda i: (0, 0)),
      ],
      out_specs=pl.BlockSpec((BM, H), lambda i: (i, 0)),
      out_shape=jax.ShapeDtypeStruct((NPAD, H), jnp.float32),
  )(degp, x, w)


def _tc_mid(degp, acc, y, b, w, wout):
  hin = y.shape[1]
  return pl.pallas_call(
      _tc_mid_body,
      grid=(GRID,),
      in_specs=[
          _degp_spec(),
          pl.BlockSpec((2, BM, hin), lambda i: (0, i, 0)),
          pl.BlockSpec((BM, hin), lambda i: (i, 0)),
          pl.BlockSpec((1, hin), lambda i: (0, 0)),
          pl.BlockSpec((hin, wout), lambda i: (0, 0)),
      ],
      out_specs=pl.BlockSpec((BM, wout), lambda i: (i, 0)),
      out_shape=jax.ShapeDtypeStruct((NPAD, wout), jnp.float32),
  )(degp, acc, y, b, w)


def _tc_last(degp, acc, y, b):
  hin = y.shape[1]
  return pl.pallas_call(
      _tc_last_body,
      grid=(GRID,),
      in_specs=[
          _degp_spec(),
          pl.BlockSpec((2, BM, hin), lambda i: (0, i, 0)),
          pl.BlockSpec((BM, hin), lambda i: (i, 0)),
          pl.BlockSpec((1, hin), lambda i: (0, 0)),
      ],
      out_specs=pl.BlockSpec((BM, hin), lambda i: (i, 0)),
      out_shape=jax.ShapeDtypeStruct((NPAD, hin), jnp.float32),
  )(degp, acc, y, b)


def kernel(x, edge_index, W1, b1, W2, b2, W3, b3):
  src = edge_index[0]
  dst = edge_index[1]
  pad_e = EPAD - E
  trash = N + jnp.arange(pad_e, dtype=jnp.int32) % (NPAD - N)
  src_p = jnp.concatenate([src, jnp.zeros((pad_e,), jnp.int32)])
  dst_p = jnp.concatenate([dst, trash])
  src_r = src_p.reshape(NTILES, NCHUNKS, CHUNK)
  dst_r = dst_p.reshape(NTILES, NCHUNKS, 1, CHUNK)

  xp = jnp.pad(x, ((0, NPAD - N), (0, 0)))
  ones16 = jnp.ones((CHUNK, OUTP), jnp.float32)
  z64 = jnp.zeros((ROWS_PER_TILE, H), jnp.float32)
  z16 = jnp.zeros((ROWS_PER_TILE, OUTP), jnp.float32)

  degp = _sc_deg_w16(ones16, dst_r, z16)

  y1 = _tc_first(degp, xp, W1)
  acc1 = _sc_scatter_w64(y1, src_r, dst_r, z64)
  y2 = _tc_mid(degp, acc1, y1, b1.reshape(1, H), W2, H)
  acc2 = _sc_scatter_w64(y2, src_r, dst_r, z64)
  w3p = jnp.pad(W3, ((0, 0), (0, OUTP - OUT)))
  y3 = _tc_mid(degp, acc2, y2, b2.reshape(1, H), w3p, OUTP)
  acc3 = _sc_scatter_w16(y3, src_r, dst_r, z16)
  b3p = jnp.pad(b3, (0, OUTP - OUT)).reshape(1, OUTP)
  out = _tc_last(degp, acc3, y3, b3p)
  return out[:N, :OUT]


# speedup vs baseline: 42.5147x; 1.1338x over previous
import functools

import jax
import jax.numpy as jnp
from jax import lax
from jax.experimental import pallas as pl
from jax.experimental.pallas import tpu as pltpu
from jax.experimental.pallas import tpu_sc as plsc

N = 10000
E = 320000
D = 128
H = 64
OUT = 10
OUTP = 16

NPAD = 10240
NTILES = 32
EPT = NPAD
EPAD = NTILES * EPT
CHUNK = 128
NCHUNKS = EPT // CHUNK
ROWS_PER_TILE = NPAD // 16
TRASH = N + 8


def _sc_scatter_fn(width):
  mesh = plsc.VectorSubcoreMesh(core_axis_name="c", subcore_axis_name="s")

  @functools.partial(
      pl.kernel,
      mesh=mesh,
      compiler_params=pltpu.CompilerParams(use_tc_tiling_on_sc=False),
      out_type=jax.ShapeDtypeStruct((2, NPAD, width), jnp.float32),
      scratch_types=[
          pltpu.VMEM((NCHUNKS, CHUNK), jnp.int32),
          pltpu.VMEM((NCHUNKS, 1, CHUNK), jnp.int32),
          pltpu.VMEM((3, CHUNK, width), jnp.float32),
          pltpu.VMEM_SHARED((NPAD, width), jnp.float32),
          pltpu.VMEM_SHARED((NPAD, width), jnp.float32),
          pltpu.SemaphoreType.DMA,
          pltpu.SemaphoreType.DMA,
      ],
  )
  def sc_scatter(table_hbm, src_hbm, dst_hbm, zeros_hbm, out_hbm,
                 src_v, dst_v, bufs, tab_sh, acc_sh, semg, sems):
    c = lax.axis_index("c")
    s = lax.axis_index("s")
    wid = s * 2 + c
    base = s * ROWS_PER_TILE
    pltpu.sync_copy(zeros_hbm, acc_sh.at[pl.ds(base, ROWS_PER_TILE)])
    pltpu.sync_copy(table_hbm.at[pl.ds(base, ROWS_PER_TILE)],
                    tab_sh.at[pl.ds(base, ROWS_PER_TILE)])
    pltpu.sync_copy(src_hbm.at[wid], src_v)
    pltpu.sync_copy(dst_hbm.at[wid], dst_v)
    plsc.subcore_barrier()

    pltpu.async_copy(tab_sh.at[src_v.at[0]], bufs.at[0], semg)
    pltpu.async_copy(tab_sh.at[src_v.at[1]], bufs.at[1], semg)

    def body(j, carry):
      par = lax.rem(j, 3)
      pltpu.make_async_copy(tab_sh.at[src_v.at[j]], bufs.at[par], semg).wait()
      pltpu.async_copy(bufs.at[par], acc_sh.at[dst_v.at[j, 0]], sems,
                       add=True)

      @pl.when(j >= 1)
      def _():
        pltpu.make_async_copy(bufs.at[par], acc_sh.at[dst_v.at[j, 0]],
                              sems).wait()

      @pl.when(j < NCHUNKS - 2)
      def _():
        pltpu.async_copy(tab_sh.at[src_v.at[j + 2]],
                         bufs.at[lax.rem(j + 2, 3)], semg)
      return carry

    lax.fori_loop(0, NCHUNKS, body, 0)
    pltpu.make_async_copy(bufs.at[0], acc_sh.at[dst_v.at[0, 0]], sems).wait()
    plsc.subcore_barrier()
    pltpu.sync_copy(acc_sh.at[pl.ds(base, ROWS_PER_TILE)],
                    out_hbm.at[c, pl.ds(base, ROWS_PER_TILE)])

  return sc_scatter


def _sc_deg_fn(width):
  mesh = plsc.VectorSubcoreMesh(core_axis_name="c", subcore_axis_name="s")

  @functools.partial(
      pl.kernel,
      mesh=mesh,
      compiler_params=pltpu.CompilerParams(use_tc_tiling_on_sc=False),
      out_type=jax.ShapeDtypeStruct((2, NPAD, width), jnp.float32),
      scratch_types=[
          pltpu.VMEM((NCHUNKS, 1, CHUNK), jnp.int32),
          pltpu.VMEM((CHUNK, width), jnp.float32),
          pltpu.VMEM_SHARED((NPAD, width), jnp.float32),
      ],
  )
  def sc_deg(ones_hbm, dst_hbm, zeros_hbm, out_hbm, dst_v, ones_v, acc_sh):
    c = lax.axis_index("c")
    s = lax.axis_index("s")
    wid = s * 2 + c
    base = s * ROWS_PER_TILE
    pltpu.sync_copy(zeros_hbm, acc_sh.at[pl.ds(base, ROWS_PER_TILE)])
    pltpu.sync_copy(ones_hbm, ones_v)
    pltpu.sync_copy(dst_hbm.at[wid], dst_v)
    plsc.subcore_barrier()

    def body(j, carry):
      pltpu.sync_copy(ones_v, acc_sh.at[dst_v.at[j, 0]], add=True)
      return carry

    lax.fori_loop(0, NCHUNKS, body, 0)
    plsc.subcore_barrier()
    pltpu.sync_copy(acc_sh.at[pl.ds(base, ROWS_PER_TILE)],
                    out_hbm.at[c, pl.ds(base, ROWS_PER_TILE)])

  return sc_deg


_sc_scatter_w64 = _sc_scatter_fn(H)
_sc_scatter_w16 = _sc_scatter_fn(OUTP)
_sc_deg_w16 = _sc_deg_fn(OUTP)


BM = 2048
BMP = BM // 2
GRID = NPAD // BM


def _tc_first_body(x_ref, w_ref, disp_ref, y_ref):
  xw = jnp.dot(x_ref[...], w_ref[...], preferred_element_type=jnp.float32)
  y_ref[...] = xw * disp_ref[...]


def _tc_mid_body(accp_ref, yp_ref, disp_ref, dispo_ref, bd_ref, wd_ref,
                 out_ref):
  agg = accp_ref[0] + accp_ref[1] + yp_ref[...]
  h = jnp.maximum(agg * disp_ref[...] + bd_ref[...], 0.0)
  out_ref[...] = jnp.dot(h, wd_ref[...],
                         preferred_element_type=jnp.float32) * dispo_ref[...]


def _tc_last_body(accp_ref, yp_ref, disp_ref, bd_ref, out_ref):
  out_ref[...] = ((accp_ref[0] + accp_ref[1] + yp_ref[...]) * disp_ref[...]
                  + bd_ref[...])


def _tc_first(x, w, disp):
  return pl.pallas_call(
      _tc_first_body,
      grid=(GRID,),
      in_specs=[
          pl.BlockSpec((BMP, 2 * D), lambda i: (i, 0)),
          pl.BlockSpec((2 * D, 128), lambda i: (0, 0)),
          pl.BlockSpec((BMP, 128), lambda i: (i, 0)),
      ],
      out_specs=pl.BlockSpec((BMP, 128), lambda i: (i, 0)),
      out_shape=jax.ShapeDtypeStruct((NPAD // 2, 128), jnp.float32),
  )(x, w, disp)


def _tc_mid(accp, yp, disp, dispo, bd, wd):
  wout = wd.shape[1]
  return pl.pallas_call(
      _tc_mid_body,
      grid=(GRID,),
      in_specs=[
          pl.BlockSpec((2, BMP, 128), lambda i: (0, i, 0)),
          pl.BlockSpec((BMP, 128), lambda i: (i, 0)),
          pl.BlockSpec((BMP, 128), lambda i: (i, 0)),
          pl.BlockSpec((BMP, wout), lambda i: (i, 0)),
          pl.BlockSpec((1, 128), lambda i: (0, 0)),
          pl.BlockSpec((128, wout), lambda i: (0, 0)),
      ],
      out_specs=pl.BlockSpec((BMP, wout), lambda i: (i, 0)),
      out_shape=jax.ShapeDtypeStruct((NPAD // 2, wout), jnp.float32),
  )(accp, yp, disp, dispo, bd, wd)


def _tc_last(accp8, yp8, disp8, bd8):
  bmp8 = BM // 8
  return pl.pallas_call(
      _tc_last_body,
      grid=(GRID,),
      in_specs=[
          pl.BlockSpec((2, bmp8, 128), lambda i: (0, i, 0)),
          pl.BlockSpec((bmp8, 128), lambda i: (i, 0)),
          pl.BlockSpec((bmp8, 128), lambda i: (i, 0)),
          pl.BlockSpec((1, 128), lambda i: (0, 0)),
      ],
      out_specs=pl.BlockSpec((bmp8, 128), lambda i: (i, 0)),
      out_shape=jax.ShapeDtypeStruct((NPAD // 8, 128), jnp.float32),
  )(accp8, yp8, disp8, bd8)


def _blockdiag(w):
  kin, kout = w.shape
  z = jnp.zeros((kin, kout), w.dtype)
  return jnp.concatenate([
      jnp.concatenate([w, z], axis=1),
      jnp.concatenate([z, w], axis=1),
  ], axis=0)


def kernel(x, edge_index, W1, b1, W2, b2, W3, b3):
  src = edge_index[0]
  dst = edge_index[1]
  pad_e = EPAD - E
  trash = N + jnp.arange(pad_e, dtype=jnp.int32) % (NPAD - N)
  src_p = jnp.concatenate([src, jnp.zeros((pad_e,), jnp.int32)])
  dst_p = jnp.concatenate([dst, trash])
  src_r = src_p.reshape(NTILES, NCHUNKS, CHUNK)
  dst_r = dst_p.reshape(NTILES, NCHUNKS, 1, CHUNK)

  xp = jnp.pad(x, ((0, NPAD - N), (0, 0)))
  ones16 = jnp.ones((CHUNK, OUTP), jnp.float32)
  z64 = jnp.zeros((ROWS_PER_TILE, H), jnp.float32)
  z16 = jnp.zeros((ROWS_PER_TILE, OUTP), jnp.float32)

  degp = _sc_deg_w16(ones16, dst_r, z16)

  deg = degp[0, :, 0] + degp[1, :, 0] + 1.0
  dis = lax.rsqrt(deg)
  disp = jnp.broadcast_to(dis[:, None], (NPAD, H)).reshape(NPAD // 2, 128)
  disp32 = jnp.broadcast_to(dis[:, None], (NPAD, OUTP)).reshape(NPAD // 2, 32)
  disp8 = jnp.broadcast_to(dis[:, None], (NPAD, OUTP)).reshape(NPAD // 8, 128)
  b1d = jnp.concatenate([b1, b1]).reshape(1, 128)
  b2d = jnp.concatenate([b2, b2]).reshape(1, 128)
  w2d = _blockdiag(W2)
  w3p = jnp.pad(W3, ((0, 0), (0, OUTP - OUT)))
  w3d = _blockdiag(w3p)
  b3p = jnp.pad(b3, (0, OUTP - OUT))
  b3d8 = jnp.tile(b3p, 8).reshape(1, 128)

  xpp = xp.reshape(NPAD // 2, 2 * D)
  w1d = _blockdiag(W1)
  y1p = _tc_first(xpp, w1d, disp)
  acc1 = _sc_scatter_w64(y1p.reshape(NPAD, H), src_r, dst_r, z64)
  y2p = _tc_mid(acc1.reshape(2, NPAD // 2, 128), y1p, disp, disp, b1d, w2d)
  acc2 = _sc_scatter_w64(y2p.reshape(NPAD, H), src_r, dst_r, z64)
  y3p = _tc_mid(acc2.reshape(2, NPAD // 2, 128), y2p, disp, disp32, b2d, w3d)
  acc3 = _sc_scatter_w16(y3p.reshape(NPAD, OUTP), src_r, dst_r, z16)
  out8 = _tc_last(acc3.reshape(2, NPAD // 8, 128),
                  y3p.reshape(NPAD // 8, 128), disp8, b3d8)
  return out8.reshape(NPAD, OUTP)[:N, :OUT]


# speedup vs baseline: 43.3097x; 1.0187x over previous
import functools

import jax
import jax.numpy as jnp
from jax import lax
from jax.experimental import pallas as pl
from jax.experimental.pallas import tpu as pltpu
from jax.experimental.pallas import tpu_sc as plsc

N = 10000
E = 320000
D = 128
H = 64
OUT = 10
OUTP = 16

NPAD = 10240
NTILES = 32
EPT = NPAD
EPAD = NTILES * EPT
CHUNK = 128
NCHUNKS = EPT // CHUNK
ROWS_PER_TILE = NPAD // 16
TRASH = N + 8


def _sc_scatter_fn(width):
  mesh = plsc.VectorSubcoreMesh(core_axis_name="c", subcore_axis_name="s")

  @functools.partial(
      pl.kernel,
      mesh=mesh,
      compiler_params=pltpu.CompilerParams(use_tc_tiling_on_sc=False),
      out_type=jax.ShapeDtypeStruct((2, NPAD, width), jnp.float32),
      scratch_types=[
          pltpu.VMEM((NCHUNKS, CHUNK), jnp.int32),
          pltpu.VMEM((NCHUNKS, 1, CHUNK), jnp.int32),
          pltpu.VMEM((3, CHUNK, width), jnp.float32),
          pltpu.VMEM_SHARED((NPAD, width), jnp.float32),
          pltpu.VMEM_SHARED((NPAD, width), jnp.float32),
          pltpu.SemaphoreType.DMA,
          pltpu.SemaphoreType.DMA,
      ],
  )
  def sc_scatter(table_hbm, src_hbm, dst_hbm, zeros_hbm, out_hbm,
                 src_v, dst_v, bufs, tab_sh, acc_sh, semg, sems):
    c = lax.axis_index("c")
    s = lax.axis_index("s")
    wid = s * 2 + c
    base = s * ROWS_PER_TILE
    pltpu.sync_copy(zeros_hbm, acc_sh.at[pl.ds(base, ROWS_PER_TILE)])
    pltpu.sync_copy(table_hbm.at[pl.ds(base, ROWS_PER_TILE)],
                    tab_sh.at[pl.ds(base, ROWS_PER_TILE)])
    pltpu.sync_copy(src_hbm.at[wid], src_v)
    pltpu.sync_copy(dst_hbm.at[wid], dst_v)
    plsc.subcore_barrier()

    pltpu.async_copy(tab_sh.at[src_v.at[0]], bufs.at[0], semg)
    pltpu.async_copy(tab_sh.at[src_v.at[1]], bufs.at[1], semg)

    def body(j, carry):
      par = lax.rem(j, 3)
      pltpu.make_async_copy(tab_sh.at[src_v.at[j]], bufs.at[par], semg).wait()
      pltpu.async_copy(bufs.at[par], acc_sh.at[dst_v.at[j, 0]], sems,
                       add=True)

      @pl.when(j >= 1)
      def _():
        pltpu.make_async_copy(bufs.at[par], acc_sh.at[dst_v.at[j, 0]],
                              sems).wait()

      @pl.when(j < NCHUNKS - 2)
      def _():
        pltpu.async_copy(tab_sh.at[src_v.at[j + 2]],
                         bufs.at[lax.rem(j + 2, 3)], semg)
      return carry

    lax.fori_loop(0, NCHUNKS, body, 0)
    pltpu.make_async_copy(bufs.at[0], acc_sh.at[dst_v.at[0, 0]], sems).wait()
    plsc.subcore_barrier()
    pltpu.sync_copy(acc_sh.at[pl.ds(base, ROWS_PER_TILE)],
                    out_hbm.at[c, pl.ds(base, ROWS_PER_TILE)])

  return sc_scatter


def _sc_deg_fn(width):
  mesh = plsc.VectorSubcoreMesh(core_axis_name="c", subcore_axis_name="s")

  @functools.partial(
      pl.kernel,
      mesh=mesh,
      compiler_params=pltpu.CompilerParams(use_tc_tiling_on_sc=False),
      out_type=jax.ShapeDtypeStruct((2, NPAD, width), jnp.float32),
      scratch_types=[
          pltpu.VMEM((NCHUNKS, 1, CHUNK), jnp.int32),
          pltpu.VMEM((CHUNK, width), jnp.float32),
          pltpu.VMEM_SHARED((NPAD, width), jnp.float32),
          pltpu.SemaphoreType.DMA,
      ],
  )
  def sc_deg(ones_hbm, dst_hbm, zeros_hbm, out_hbm, dst_v, ones_v, acc_sh,
             sems):
    c = lax.axis_index("c")
    s = lax.axis_index("s")
    wid = s * 2 + c
    base = s * ROWS_PER_TILE
    pltpu.sync_copy(zeros_hbm, acc_sh.at[pl.ds(base, ROWS_PER_TILE)])
    pltpu.sync_copy(ones_hbm, ones_v)
    pltpu.sync_copy(dst_hbm.at[wid], dst_v)
    plsc.subcore_barrier()

    def body(j, carry):
      pltpu.async_copy(ones_v, acc_sh.at[dst_v.at[j, 0]], sems, add=True)

      @pl.when(j >= 3)
      def _():
        pltpu.make_async_copy(ones_v, acc_sh.at[dst_v.at[j, 0]], sems).wait()
      return carry

    lax.fori_loop(0, NCHUNKS, body, 0)
    for _ in range(3):
      pltpu.make_async_copy(ones_v, acc_sh.at[dst_v.at[0, 0]], sems).wait()
    plsc.subcore_barrier()
    pltpu.sync_copy(acc_sh.at[pl.ds(base, ROWS_PER_TILE)],
                    out_hbm.at[c, pl.ds(base, ROWS_PER_TILE)])

  return sc_deg


_sc_scatter_w64 = _sc_scatter_fn(H)
_sc_scatter_w16 = _sc_scatter_fn(OUTP)
_sc_deg_w16 = _sc_deg_fn(OUTP)


BM = 2048
BMP = BM // 2
GRID = NPAD // BM


def _tc_first_body(x_ref, w_ref, disp_ref, y_ref):
  xw = jnp.dot(x_ref[...], w_ref[...], preferred_element_type=jnp.float32)
  y_ref[...] = xw * disp_ref[...]


def _tc_mid_body(accp_ref, yp_ref, disp_ref, dispo_ref, bd_ref, wd_ref,
                 out_ref):
  agg = accp_ref[0] + accp_ref[1] + yp_ref[...]
  h = jnp.maximum(agg * disp_ref[...] + bd_ref[...], 0.0)
  out_ref[...] = jnp.dot(h, wd_ref[...],
                         preferred_element_type=jnp.float32) * dispo_ref[...]


def _tc_last_body(accp_ref, yp_ref, disp_ref, bd_ref, out_ref):
  out_ref[...] = ((accp_ref[0] + accp_ref[1] + yp_ref[...]) * disp_ref[...]
                  + bd_ref[...])


def _tc_first(x, w, disp):
  return pl.pallas_call(
      _tc_first_body,
      grid=(GRID,),
      in_specs=[
          pl.BlockSpec((BMP, 2 * D), lambda i: (i, 0)),
          pl.BlockSpec((2 * D, 128), lambda i: (0, 0)),
          pl.BlockSpec((BMP, 128), lambda i: (i, 0)),
      ],
      out_specs=pl.BlockSpec((BMP, 128), lambda i: (i, 0)),
      out_shape=jax.ShapeDtypeStruct((NPAD // 2, 128), jnp.float32),
  )(x, w, disp)


def _tc_mid(accp, yp, disp, dispo, bd, wd):
  wout = wd.shape[1]
  return pl.pallas_call(
      _tc_mid_body,
      grid=(GRID,),
      in_specs=[
          pl.BlockSpec((2, BMP, 128), lambda i: (0, i, 0)),
          pl.BlockSpec((BMP, 128), lambda i: (i, 0)),
          pl.BlockSpec((BMP, 128), lambda i: (i, 0)),
          pl.BlockSpec((BMP, wout), lambda i: (i, 0)),
          pl.BlockSpec((1, 128), lambda i: (0, 0)),
          pl.BlockSpec((128, wout), lambda i: (0, 0)),
      ],
      out_specs=pl.BlockSpec((BMP, wout), lambda i: (i, 0)),
      out_shape=jax.ShapeDtypeStruct((NPAD // 2, wout), jnp.float32),
  )(accp, yp, disp, dispo, bd, wd)


def _tc_last(accp8, yp8, disp8, bd8):
  bmp8 = BM // 8
  return pl.pallas_call(
      _tc_last_body,
      grid=(GRID,),
      in_specs=[
          pl.BlockSpec((2, bmp8, 128), lambda i: (0, i, 0)),
          pl.BlockSpec((bmp8, 128), lambda i: (i, 0)),
          pl.BlockSpec((bmp8, 128), lambda i: (i, 0)),
          pl.BlockSpec((1, 128), lambda i: (0, 0)),
      ],
      out_specs=pl.BlockSpec((bmp8, 128), lambda i: (i, 0)),
      out_shape=jax.ShapeDtypeStruct((NPAD // 8, 128), jnp.float32),
  )(accp8, yp8, disp8, bd8)


def _blockdiag(w):
  kin, kout = w.shape
  z = jnp.zeros((kin, kout), w.dtype)
  return jnp.concatenate([
      jnp.concatenate([w, z], axis=1),
      jnp.concatenate([z, w], axis=1),
  ], axis=0)


def kernel(x, edge_index, W1, b1, W2, b2, W3, b3):
  src = edge_index[0]
  dst = edge_index[1]
  pad_e = EPAD - E
  trash = N + jnp.arange(pad_e, dtype=jnp.int32) % (NPAD - N)
  src_p = jnp.concatenate([src, jnp.zeros((pad_e,), jnp.int32)])
  dst_p = jnp.concatenate([dst, trash])
  src_r = src_p.reshape(NTILES, NCHUNKS, CHUNK)
  dst_r = dst_p.reshape(NTILES, NCHUNKS, 1, CHUNK)

  ones16 = jnp.ones((CHUNK, OUTP), jnp.float32)
  z64 = jnp.zeros((ROWS_PER_TILE, H), jnp.float32)
  z16 = jnp.zeros((ROWS_PER_TILE, OUTP), jnp.float32)

  degp = _sc_deg_w16(ones16, dst_r, z16)

  deg = degp[0, :, 0] + degp[1, :, 0] + 1.0
  dis = lax.rsqrt(deg)
  disp = jnp.broadcast_to(dis[:, None], (NPAD, H)).reshape(NPAD // 2, 128)
  disp32 = jnp.broadcast_to(dis[:, None], (NPAD, OUTP)).reshape(NPAD // 2, 32)
  disp8 = jnp.broadcast_to(dis[:, None], (NPAD, OUTP)).reshape(NPAD // 8, 128)
  b1d = jnp.concatenate([b1, b1]).reshape(1, 128)
  b2d = jnp.concatenate([b2, b2]).reshape(1, 128)
  w2d = _blockdiag(W2)
  w3p = jnp.pad(W3, ((0, 0), (0, OUTP - OUT)))
  w3d = _blockdiag(w3p)
  b3p = jnp.pad(b3, (0, OUTP - OUT))
  b3d8 = jnp.tile(b3p, 8).reshape(1, 128)

  xpp = jnp.pad(x.reshape(N // 2, 2 * D), ((0, (NPAD - N) // 2), (0, 0)))
  w1d = _blockdiag(W1)
  y1p = _tc_first(xpp, w1d, disp)
  acc1 = _sc_scatter_w64(y1p.reshape(NPAD, H), src_r, dst_r, z64)
  y2p = _tc_mid(acc1.reshape(2, NPAD // 2, 128), y1p, disp, disp, b1d, w2d)
  acc2 = _sc_scatter_w64(y2p.reshape(NPAD, H), src_r, dst_r, z64)
  y3p = _tc_mid(acc2.reshape(2, NPAD // 2, 128), y2p, disp, disp32, b2d, w3d)
  acc3 = _sc_scatter_w16(y3p.reshape(NPAD, OUTP), src_r, dst_r, z16)
  out8 = _tc_last(acc3.reshape(2, NPAD // 8, 128),
                  y3p.reshape(NPAD // 8, 128), disp8, b3d8)
  return out8.reshape(NPAD, OUTP)[:N, :OUT]


# speedup vs baseline: 43.5616x; 1.0058x over previous
import functools

import jax
import jax.numpy as jnp
from jax import lax
from jax.experimental import pallas as pl
from jax.experimental.pallas import tpu as pltpu
from jax.experimental.pallas import tpu_sc as plsc

N = 10000
E = 320000
D = 128
H = 64
OUT = 10
OUTP = 16

NPAD = 10240
NTILES = 32
EPT = NPAD
EPAD = NTILES * EPT
CHUNK = 128
NCHUNKS = EPT // CHUNK
ROWS_PER_TILE = NPAD // 16
TRASH = N + 8


def _sc_scatter_fn(width):
  mesh = plsc.VectorSubcoreMesh(core_axis_name="c", subcore_axis_name="s")

  @functools.partial(
      pl.kernel,
      mesh=mesh,
      compiler_params=pltpu.CompilerParams(use_tc_tiling_on_sc=False),
      out_type=jax.ShapeDtypeStruct((2, NPAD, width), jnp.float32),
      scratch_types=[
          pltpu.VMEM((NCHUNKS, CHUNK), jnp.int32),
          pltpu.VMEM((NCHUNKS, 1, CHUNK), jnp.int32),
          pltpu.VMEM((3, CHUNK, width), jnp.float32),
          pltpu.VMEM_SHARED((NPAD, width), jnp.float32),
          pltpu.VMEM_SHARED((NPAD, width), jnp.float32),
          pltpu.SemaphoreType.DMA,
          pltpu.SemaphoreType.DMA,
      ],
  )
  def sc_scatter(table_hbm, src_hbm, dst_hbm, zeros_hbm, out_hbm,
                 src_v, dst_v, bufs, tab_sh, acc_sh, semg, sems):
    c = lax.axis_index("c")
    s = lax.axis_index("s")
    wid = s * 2 + c
    base = s * ROWS_PER_TILE
    pltpu.sync_copy(zeros_hbm, acc_sh.at[pl.ds(base, ROWS_PER_TILE)])
    pltpu.sync_copy(table_hbm.at[pl.ds(base, ROWS_PER_TILE)],
                    tab_sh.at[pl.ds(base, ROWS_PER_TILE)])
    pltpu.sync_copy(src_hbm.at[wid], src_v)
    pltpu.sync_copy(dst_hbm.at[wid], dst_v)
    plsc.subcore_barrier()

    pltpu.async_copy(tab_sh.at[src_v.at[0]], bufs.at[0], semg)
    pltpu.async_copy(tab_sh.at[src_v.at[1]], bufs.at[1], semg)

    def body(j, carry):
      par = lax.rem(j, 3)
      pltpu.make_async_copy(tab_sh.at[src_v.at[j]], bufs.at[par], semg).wait()
      pltpu.async_copy(bufs.at[par], acc_sh.at[dst_v.at[j, 0]], sems,
                       add=True)

      @pl.when(j >= 1)
      def _():
        pltpu.make_async_copy(bufs.at[par], acc_sh.at[dst_v.at[j, 0]],
                              sems).wait()

      @pl.when(j < NCHUNKS - 2)
      def _():
        pltpu.async_copy(tab_sh.at[src_v.at[j + 2]],
                         bufs.at[lax.rem(j + 2, 3)], semg)
      return carry

    lax.fori_loop(0, NCHUNKS, body, 0)
    pltpu.make_async_copy(bufs.at[0], acc_sh.at[dst_v.at[0, 0]], sems).wait()
    plsc.subcore_barrier()
    pltpu.sync_copy(acc_sh.at[pl.ds(base, ROWS_PER_TILE)],
                    out_hbm.at[c, pl.ds(base, ROWS_PER_TILE)])

  return sc_scatter


def _sc_deg_fn(width):
  mesh = plsc.VectorSubcoreMesh(core_axis_name="c", subcore_axis_name="s")

  @functools.partial(
      pl.kernel,
      mesh=mesh,
      compiler_params=pltpu.CompilerParams(use_tc_tiling_on_sc=False),
      out_type=jax.ShapeDtypeStruct((2, NPAD, width), jnp.float32),
      scratch_types=[
          pltpu.VMEM((NCHUNKS, 1, CHUNK), jnp.int32),
          pltpu.VMEM((CHUNK, width), jnp.float32),
          pltpu.VMEM_SHARED((NPAD, width), jnp.float32),
          pltpu.SemaphoreType.DMA,
      ],
  )
  def sc_deg(ones_hbm, dst_hbm, zeros_hbm, out_hbm, dst_v, ones_v, acc_sh,
             sems):
    c = lax.axis_index("c")
    s = lax.axis_index("s")
    wid = s * 2 + c
    base = s * ROWS_PER_TILE
    pltpu.sync_copy(zeros_hbm, acc_sh.at[pl.ds(base, ROWS_PER_TILE)])
    pltpu.sync_copy(ones_hbm, ones_v)
    pltpu.sync_copy(dst_hbm.at[wid], dst_v)
    plsc.subcore_barrier()

    def body(j, carry):
      pltpu.async_copy(ones_v, acc_sh.at[dst_v.at[j, 0]], sems, add=True)

      @pl.when(j >= 3)
      def _():
        pltpu.make_async_copy(ones_v, acc_sh.at[dst_v.at[j, 0]], sems).wait()
      return carry

    lax.fori_loop(0, NCHUNKS, body, 0)
    for _ in range(3):
      pltpu.make_async_copy(ones_v, acc_sh.at[dst_v.at[0, 0]], sems).wait()
    plsc.subcore_barrier()
    pltpu.sync_copy(acc_sh.at[pl.ds(base, ROWS_PER_TILE)],
                    out_hbm.at[c, pl.ds(base, ROWS_PER_TILE)])

  return sc_deg


_sc_scatter_w64 = _sc_scatter_fn(H)
_sc_scatter_w16 = _sc_scatter_fn(OUTP)
_sc_deg_w16 = _sc_deg_fn(OUTP)


BM = 2048
BMP = BM // 2
GRID = NPAD // BM


def _tc_first_body(x_ref, w_ref, disp_ref, y_ref):
  xw = jnp.dot(x_ref[...], w_ref[...], preferred_element_type=jnp.float32)
  y_ref[...] = xw * disp_ref[...]


def _tc_mid_body(accp_ref, yp_ref, disp_ref, dispo_ref, bd_ref, wd_ref,
                 out_ref):
  agg = accp_ref[0] + accp_ref[1] + yp_ref[...]
  h = jnp.maximum(agg * disp_ref[...] + bd_ref[...], 0.0)
  out_ref[...] = jnp.dot(h, wd_ref[...],
                         preferred_element_type=jnp.float32) * dispo_ref[...]


def _tc_last_body(accp_ref, yp_ref, disp_ref, bd_ref, out_ref):
  out_ref[...] = ((accp_ref[0] + accp_ref[1] + yp_ref[...]) * disp_ref[...]
                  + bd_ref[...])


def _tc_first(x, w, disp):
  return pl.pallas_call(
      _tc_first_body,
      grid=(GRID,),
      in_specs=[
          pl.BlockSpec((BMP, 2 * D), lambda i: (i, 0)),
          pl.BlockSpec((2 * D, 128), lambda i: (0, 0)),
          pl.BlockSpec((BMP, 128), lambda i: (i, 0)),
      ],
      out_specs=pl.BlockSpec((BMP, 128), lambda i: (i, 0)),
      out_shape=jax.ShapeDtypeStruct((NPAD // 2, 128), jnp.float32),
  )(x, w, disp)


def _tc_mid(accp, yp, disp, dispo, bd, wd):
  wout = wd.shape[1]
  return pl.pallas_call(
      _tc_mid_body,
      grid=(GRID,),
      in_specs=[
          pl.BlockSpec((2, BMP, 128), lambda i: (0, i, 0)),
          pl.BlockSpec((BMP, 128), lambda i: (i, 0)),
          pl.BlockSpec((BMP, 128), lambda i: (i, 0)),
          pl.BlockSpec((BMP, wout), lambda i: (i, 0)),
          pl.BlockSpec((1, 128), lambda i: (0, 0)),
          pl.BlockSpec((128, wout), lambda i: (0, 0)),
      ],
      out_specs=pl.BlockSpec((BMP, wout), lambda i: (i, 0)),
      out_shape=jax.ShapeDtypeStruct((NPAD // 2, wout), jnp.float32),
  )(accp, yp, disp, dispo, bd, wd)


def _tc_last(accp8, yp8, disp8, bd8):
  bmp8 = BM // 8
  return pl.pallas_call(
      _tc_last_body,
      grid=(GRID,),
      in_specs=[
          pl.BlockSpec((2, bmp8, 128), lambda i: (0, i, 0)),
          pl.BlockSpec((bmp8, 128), lambda i: (i, 0)),
          pl.BlockSpec((bmp8, 128), lambda i: (i, 0)),
          pl.BlockSpec((1, 128), lambda i: (0, 0)),
      ],
      out_specs=pl.BlockSpec((bmp8, 128), lambda i: (i, 0)),
      out_shape=jax.ShapeDtypeStruct((NPAD // 8, 128), jnp.float32),
  )(accp8, yp8, disp8, bd8)


def _blockdiag(w):
  kin, kout = w.shape
  z = jnp.zeros((kin, kout), w.dtype)
  return jnp.concatenate([
      jnp.concatenate([w, z], axis=1),
      jnp.concatenate([z, w], axis=1),
  ], axis=0)


def kernel(x, edge_index, W1, b1, W2, b2, W3, b3):
  ei_flat = edge_index.reshape(2 * E)
  src = ei_flat[:E]
  dst = ei_flat[E:]
  pad_e = EPAD - E
  trash = N + jnp.arange(pad_e, dtype=jnp.int32) % (NPAD - N)
  src_p = jnp.concatenate([src, jnp.zeros((pad_e,), jnp.int32)])
  dst_p = jnp.concatenate([dst, trash])
  src_r = src_p.reshape(NTILES, NCHUNKS, CHUNK)
  dst_r = dst_p.reshape(NTILES, NCHUNKS, 1, CHUNK)

  ones16 = jnp.ones((CHUNK, OUTP), jnp.float32)
  z64 = jnp.zeros((ROWS_PER_TILE, H), jnp.float32)
  z16 = jnp.zeros((ROWS_PER_TILE, OUTP), jnp.float32)

  degp = _sc_deg_w16(ones16, dst_r, z16)

  deg = degp[0, :, 0] + degp[1, :, 0] + 1.0
  dis = lax.rsqrt(deg)
  disp = jnp.broadcast_to(dis[:, None], (NPAD, H)).reshape(NPAD // 2, 128)
  disp32 = jnp.broadcast_to(dis[:, None], (NPAD, OUTP)).reshape(NPAD // 2, 32)
  disp8 = jnp.broadcast_to(dis[:, None], (NPAD, OUTP)).reshape(NPAD // 8, 128)
  b1d = jnp.concatenate([b1, b1]).reshape(1, 128)
  b2d = jnp.concatenate([b2, b2]).reshape(1, 128)
  w2d = _blockdiag(W2)
  w3p = jnp.pad(W3, ((0, 0), (0, OUTP - OUT)))
  w3d = _blockdiag(w3p)
  b3p = jnp.pad(b3, (0, OUTP - OUT))
  b3d8 = jnp.tile(b3p, 8).reshape(1, 128)

  xpp = jnp.pad(x.reshape(N // 2, 2 * D), ((0, (NPAD - N) // 2), (0, 0)))
  w1d = _blockdiag(W1)
  y1p = _tc_first(xpp, w1d, disp)
  acc1 = _sc_scatter_w64(y1p.reshape(NPAD, H), src_r, dst_r, z64)
  y2p = _tc_mid(acc1.reshape(2, NPAD // 2, 128), y1p, disp, disp, b1d, w2d)
  acc2 = _sc_scatter_w64(y2p.reshape(NPAD, H), src_r, dst_r, z64)
  y3p = _tc_mid(acc2.reshape(2, NPAD // 2, 128), y2p, disp, disp32, b2d, w3d)
  acc3 = _sc_scatter_w16(y3p.reshape(NPAD, OUTP), src_r, dst_r, z16)
  out8 = _tc_last(acc3.reshape(2, NPAD // 8, 128),
                  y3p.reshape(NPAD // 8, 128), disp8, b3d8)
  return out8.reshape(NPAD, OUTP)[:N, :OUT]


# speedup vs baseline: 43.5707x; 1.0002x over previous
import functools

import jax
import jax.numpy as jnp
from jax import lax
from jax.experimental import pallas as pl
from jax.experimental.pallas import tpu as pltpu
from jax.experimental.pallas import tpu_sc as plsc

N = 10000
E = 320000
D = 128
H = 64
OUT = 10
OUTP = 16

NPAD = 10240
NTILES = 32
EPT = NPAD
EPAD = NTILES * EPT
CHUNK = 128
NCHUNKS = EPT // CHUNK
ROWS_PER_TILE = NPAD // 16
TRASH = N + 8


def _sc_scatter_fn(width):
  mesh = plsc.VectorSubcoreMesh(core_axis_name="c", subcore_axis_name="s")

  @functools.partial(
      pl.kernel,
      mesh=mesh,
      compiler_params=pltpu.CompilerParams(use_tc_tiling_on_sc=False),
      out_type=jax.ShapeDtypeStruct((2, NPAD, width), jnp.float32),
      scratch_types=[
          pltpu.VMEM((NCHUNKS, CHUNK), jnp.int32),
          pltpu.VMEM((NCHUNKS, 1, CHUNK), jnp.int32),
          pltpu.VMEM((3, CHUNK, width), jnp.float32),
          pltpu.VMEM_SHARED((NPAD, width), jnp.float32),
          pltpu.VMEM_SHARED((NPAD, width), jnp.float32),
          pltpu.SemaphoreType.DMA,
          pltpu.SemaphoreType.DMA,
      ],
  )
  def sc_scatter(table_hbm, src_hbm, dst_hbm, zeros_hbm, out_hbm,
                 src_v, dst_v, bufs, tab_sh, acc_sh, semg, sems):
    c = lax.axis_index("c")
    s = lax.axis_index("s")
    wid = s * 2 + c
    base = s * ROWS_PER_TILE
    pltpu.sync_copy(zeros_hbm, acc_sh.at[pl.ds(base, ROWS_PER_TILE)])
    pltpu.sync_copy(table_hbm.at[pl.ds(base, ROWS_PER_TILE)],
                    tab_sh.at[pl.ds(base, ROWS_PER_TILE)])
    pltpu.sync_copy(src_hbm.at[wid], src_v)
    pltpu.sync_copy(dst_hbm.at[wid], dst_v)
    plsc.subcore_barrier()

    pltpu.async_copy(tab_sh.at[src_v.at[0]], bufs.at[0], semg)
    pltpu.async_copy(tab_sh.at[src_v.at[1]], bufs.at[1], semg)

    def body(j, carry):
      par = lax.rem(j, 3)
      pltpu.make_async_copy(tab_sh.at[src_v.at[j]], bufs.at[par], semg).wait()
      pltpu.async_copy(bufs.at[par], acc_sh.at[dst_v.at[j, 0]], sems,
                       add=True)

      @pl.when(j >= 1)
      def _():
        pltpu.make_async_copy(bufs.at[par], acc_sh.at[dst_v.at[j, 0]],
                              sems).wait()

      @pl.when(j < NCHUNKS - 2)
      def _():
        pltpu.async_copy(tab_sh.at[src_v.at[j + 2]],
                         bufs.at[lax.rem(j + 2, 3)], semg)
      return carry

    lax.fori_loop(0, NCHUNKS, body, 0, unroll=2)
    pltpu.make_async_copy(bufs.at[0], acc_sh.at[dst_v.at[0, 0]], sems).wait()
    plsc.subcore_barrier()
    pltpu.sync_copy(acc_sh.at[pl.ds(base, ROWS_PER_TILE)],
                    out_hbm.at[c, pl.ds(base, ROWS_PER_TILE)])

  return sc_scatter


def _sc_deg_fn(width):
  mesh = plsc.VectorSubcoreMesh(core_axis_name="c", subcore_axis_name="s")

  @functools.partial(
      pl.kernel,
      mesh=mesh,
      compiler_params=pltpu.CompilerParams(use_tc_tiling_on_sc=False),
      out_type=jax.ShapeDtypeStruct((2, NPAD, width), jnp.float32),
      scratch_types=[
          pltpu.VMEM((NCHUNKS, 1, CHUNK), jnp.int32),
          pltpu.VMEM((CHUNK, width), jnp.float32),
          pltpu.VMEM_SHARED((NPAD, width), jnp.float32),
          pltpu.SemaphoreType.DMA,
      ],
  )
  def sc_deg(ones_hbm, dst_hbm, zeros_hbm, out_hbm, dst_v, ones_v, acc_sh,
             sems):
    c = lax.axis_index("c")
    s = lax.axis_index("s")
    wid = s * 2 + c
    base = s * ROWS_PER_TILE
    pltpu.sync_copy(zeros_hbm, acc_sh.at[pl.ds(base, ROWS_PER_TILE)])
    pltpu.sync_copy(ones_hbm, ones_v)
    pltpu.sync_copy(dst_hbm.at[wid], dst_v)
    plsc.subcore_barrier()

    def body(j, carry):
      pltpu.async_copy(ones_v, acc_sh.at[dst_v.at[j, 0]], sems, add=True)

      @pl.when(j >= 3)
      def _():
        pltpu.make_async_copy(ones_v, acc_sh.at[dst_v.at[j, 0]], sems).wait()
      return carry

    lax.fori_loop(0, NCHUNKS, body, 0, unroll=2)
    for _ in range(3):
      pltpu.make_async_copy(ones_v, acc_sh.at[dst_v.at[0, 0]], sems).wait()
    plsc.subcore_barrier()
    pltpu.sync_copy(acc_sh.at[pl.ds(base, ROWS_PER_TILE)],
                    out_hbm.at[c, pl.ds(base, ROWS_PER_TILE)])

  return sc_deg


_sc_scatter_w64 = _sc_scatter_fn(H)
_sc_scatter_w16 = _sc_scatter_fn(OUTP)
_sc_deg_w16 = _sc_deg_fn(OUTP)


BM = 2048
BMP = BM // 2
GRID = NPAD // BM


def _tc_first_body(x_ref, w_ref, disp_ref, y_ref):
  xw = jnp.dot(x_ref[...], w_ref[...], preferred_element_type=jnp.float32)
  y_ref[...] = xw * disp_ref[...]


def _tc_mid_body(accp_ref, yp_ref, disp_ref, dispo_ref, bd_ref, wd_ref,
                 out_ref):
  agg = accp_ref[0] + accp_ref[1] + yp_ref[...]
  h = jnp.maximum(agg * disp_ref[...] + bd_ref[...], 0.0)
  out_ref[...] = jnp.dot(h, wd_ref[...],
                         preferred_element_type=jnp.float32) * dispo_ref[...]


def _tc_last_body(accp_ref, yp_ref, disp_ref, bd_ref, out_ref):
  out_ref[...] = ((accp_ref[0] + accp_ref[1] + yp_ref[...]) * disp_ref[...]
                  + bd_ref[...])


def _tc_first(x, w, disp):
  return pl.pallas_call(
      _tc_first_body,
      grid=(GRID,),
      in_specs=[
          pl.BlockSpec((BMP, 2 * D), lambda i: (i, 0)),
          pl.BlockSpec((2 * D, 128), lambda i: (0, 0)),
          pl.BlockSpec((BMP, 128), lambda i: (i, 0)),
      ],
      out_specs=pl.BlockSpec((BMP, 128), lambda i: (i, 0)),
      out_shape=jax.ShapeDtypeStruct((NPAD // 2, 128), jnp.float32),
  )(x, w, disp)


def _tc_mid(accp, yp, disp, dispo, bd, wd):
  wout = wd.shape[1]
  return pl.pallas_call(
      _tc_mid_body,
      grid=(GRID,),
      in_specs=[
          pl.BlockSpec((2, BMP, 128), lambda i: (0, i, 0)),
          pl.BlockSpec((BMP, 128), lambda i: (i, 0)),
          pl.BlockSpec((BMP, 128), lambda i: (i, 0)),
          pl.BlockSpec((BMP, wout), lambda i: (i, 0)),
          pl.BlockSpec((1, 128), lambda i: (0, 0)),
          pl.BlockSpec((128, wout), lambda i: (0, 0)),
      ],
      out_specs=pl.BlockSpec((BMP, wout), lambda i: (i, 0)),
      out_shape=jax.ShapeDtypeStruct((NPAD // 2, wout), jnp.float32),
  )(accp, yp, disp, dispo, bd, wd)


def _tc_last(accp8, yp8, disp8, bd8):
  bmp8 = BM // 8
  return pl.pallas_call(
      _tc_last_body,
      grid=(GRID,),
      in_specs=[
          pl.BlockSpec((2, bmp8, 128), lambda i: (0, i, 0)),
          pl.BlockSpec((bmp8, 128), lambda i: (i, 0)),
          pl.BlockSpec((bmp8, 128), lambda i: (i, 0)),
          pl.BlockSpec((1, 128), lambda i: (0, 0)),
      ],
      out_specs=pl.BlockSpec((bmp8, 128), lambda i: (i, 0)),
      out_shape=jax.ShapeDtypeStruct((NPAD // 8, 128), jnp.float32),
  )(accp8, yp8, disp8, bd8)


def _blockdiag(w):
  kin, kout = w.shape
  z = jnp.zeros((kin, kout), w.dtype)
  return jnp.concatenate([
      jnp.concatenate([w, z], axis=1),
      jnp.concatenate([z, w], axis=1),
  ], axis=0)


def kernel(x, edge_index, W1, b1, W2, b2, W3, b3):
  ei_flat = edge_index.reshape(2 * E)
  src = ei_flat[:E]
  dst = ei_flat[E:]
  pad_e = EPAD - E
  trash = N + jnp.arange(pad_e, dtype=jnp.int32) % (NPAD - N)
  src_p = jnp.concatenate([src, jnp.zeros((pad_e,), jnp.int32)])
  dst_p = jnp.concatenate([dst, trash])
  src_r = src_p.reshape(NTILES, NCHUNKS, CHUNK)
  dst_r = dst_p.reshape(NTILES, NCHUNKS, 1, CHUNK)

  ones16 = jnp.ones((CHUNK, OUTP), jnp.float32)
  z64 = jnp.zeros((ROWS_PER_TILE, H), jnp.float32)
  z16 = jnp.zeros((ROWS_PER_TILE, OUTP), jnp.float32)

  degp = _sc_deg_w16(ones16, dst_r, z16)

  deg = degp[0, :, 0] + degp[1, :, 0] + 1.0
  dis = lax.rsqrt(deg)
  disp = jnp.broadcast_to(dis[:, None], (NPAD, H)).reshape(NPAD // 2, 128)
  disp32 = jnp.broadcast_to(dis[:, None], (NPAD, OUTP)).reshape(NPAD // 2, 32)
  disp8 = jnp.broadcast_to(dis[:, None], (NPAD, OUTP)).reshape(NPAD // 8, 128)
  b1d = jnp.concatenate([b1, b1]).reshape(1, 128)
  b2d = jnp.concatenate([b2, b2]).reshape(1, 128)
  w2d = _blockdiag(W2)
  w3p = jnp.pad(W3, ((0, 0), (0, OUTP - OUT)))
  w3d = _blockdiag(w3p)
  b3p = jnp.pad(b3, (0, OUTP - OUT))
  b3d8 = jnp.tile(b3p, 8).reshape(1, 128)

  xpp = jnp.pad(x.reshape(N // 2, 2 * D), ((0, (NPAD - N) // 2), (0, 0)))
  w1d = _blockdiag(W1)
  y1p = _tc_first(xpp, w1d, disp)
  acc1 = _sc_scatter_w64(y1p.reshape(NPAD, H), src_r, dst_r, z64)
  y2p = _tc_mid(acc1.reshape(2, NPAD // 2, 128), y1p, disp, disp, b1d, w2d)
  acc2 = _sc_scatter_w64(y2p.reshape(NPAD, H), src_r, dst_r, z64)
  y3p = _tc_mid(acc2.reshape(2, NPAD // 2, 128), y2p, disp, disp32, b2d, w3d)
  acc3 = _sc_scatter_w16(y3p.reshape(NPAD, OUTP), src_r, dst_r, z16)
  out8 = _tc_last(acc3.reshape(2, NPAD // 8, 128),
                  y3p.reshape(NPAD // 8, 128), disp8, b3d8)
  return out8.reshape(NPAD, OUTP)[:N, :OUT]
